# Initial kernel scaffold; baseline (speedup 1.0000x reference)
import functools

import jax
import jax.numpy as jnp
from jax import lax
from jax.experimental import pallas as pl
from jax.experimental.pallas import tpu as pltpu
from jax.experimental.pallas import tpu_sc as plsc

D = 256
N_NODES = 10000
N_EDGES = 160000
NC = 2
NS = 16
L = 16

HALF = N_NODES // NC
RPT = 313
HALF_P = NS * RPT
TRASH = HALF
CW = 16

_mesh = plsc.VectorSubcoreMesh(
    core_axis_name="c", subcore_axis_name="s", num_cores=NC, num_subcores=NS)

f32 = jnp.float32
i32 = jnp.int32


def _leaky(x):
  return jnp.where(x > 0, x, 0.01 * x)


def _ln(x, g, b):
  mu = jnp.mean(x, axis=-1, keepdims=True)
  var = jnp.mean((x - mu) ** 2, axis=-1, keepdims=True)
  return (x - mu) * lax.rsqrt(var + 1e-5) * g + b


def _tc_in_proj_body(x_ref, w_ref, b_ref, wcat_ref, h_ref, p_ref):
  h = _leaky(jnp.dot(x_ref[...], w_ref[...], preferred_element_type=f32)
             + b_ref[...])
  h_ref[...] = h
  p_ref[...] = jnp.dot(h, wcat_ref[...], preferred_element_type=f32)


def _tc_in_proj(x, w, b, wcat):
  n = x.shape[0]
  br = 1000
  return pl.pallas_call(
      _tc_in_proj_body,
      grid=(n // br,),
      in_specs=[
          pl.BlockSpec((br, D), lambda i: (i, 0)),
          pl.BlockSpec((D, D), lambda i: (0, 0)),
          pl.BlockSpec((1, D), lambda i: (0, 0)),
          pl.BlockSpec((D, 2 * D), lambda i: (0, 0)),
      ],
      out_specs=[
          pl.BlockSpec((br, D), lambda i: (i, 0)),
          pl.BlockSpec((br, 2 * D), lambda i: (i, 0)),
      ],
      out_shape=[
          jax.ShapeDtypeStruct((n, D), f32),
          jax.ShapeDtypeStruct((n, 2 * D), f32),
      ],
  )(x, w, b, wcat)


def _tc_matmul_bias_body(x_ref, w_ref, b_ref, o_ref):
  o_ref[...] = (jnp.dot(x_ref[...], w_ref[...], preferred_element_type=f32)
                + b_ref[...])


def _tc_matmul_bias(x, w, b):
  n, k = x.shape
  m = w.shape[1]
  return pl.pallas_call(
      _tc_matmul_bias_body,
      grid=(1,),
      in_specs=[
          pl.BlockSpec((n, k), lambda i: (0, 0)),
          pl.BlockSpec((k, m), lambda i: (0, 0)),
          pl.BlockSpec((1, m), lambda i: (0, 0)),
      ],
      out_specs=pl.BlockSpec((n, m), lambda i: (0, 0)),
      out_shape=jax.ShapeDtypeStruct((n, m), f32),
  )(x, w, b)


def _tc_node_update_body(agg_ref, cnt_ref, hp_ref, g_ref, b_ref, wcat_ref,
                         h_ref, p_ref):
  cnt = jnp.maximum(cnt_ref[...], 1.0)
  h = _ln(_leaky(agg_ref[...] / cnt) + hp_ref[...], g_ref[...], b_ref[...])
  h_ref[...] = h
  p_ref[...] = jnp.dot(h, wcat_ref[...], preferred_element_type=f32)


def _tc_node_update(agg, cnt, h_prev, g, b, wcat):
  n = agg.shape[0]
  m = wcat.shape[1]
  br = 500
  return pl.pallas_call(
      _tc_node_update_body,
      grid=(n // br,),
      in_specs=[
          pl.BlockSpec((br, D), lambda i: (i, 0)),
          pl.BlockSpec((br, 1), lambda i: (i, 0)),
          pl.BlockSpec((br, D), lambda i: (i, 0)),
          pl.BlockSpec((1, D), lambda i: (0, 0)),
          pl.BlockSpec((1, D), lambda i: (0, 0)),
          pl.BlockSpec((D, m), lambda i: (0, 0)),
      ],
      out_specs=[
          pl.BlockSpec((br, D), lambda i: (i, 0)),
          pl.BlockSpec((br, m), lambda i: (i, 0)),
      ],
      out_shape=[
          jax.ShapeDtypeStruct((n, D), f32),
          jax.ShapeDtypeStruct((n, m), f32),
      ],
  )(agg, cnt, h_prev, g, b, wcat)


def _tc_node_final_body(agg_ref, cnt_ref, hp_ref, g_ref, b_ref, h_ref):
  cnt = jnp.maximum(cnt_ref[...], 1.0)
  h_ref[...] = _ln(_leaky(agg_ref[...] / cnt) + hp_ref[...],
                   g_ref[...], b_ref[...])


def _tc_node_final(agg, cnt, h_prev, g, b):
  n = agg.shape[0]
  br = 1000
  return pl.pallas_call(
      _tc_node_final_body,
      grid=(n // br,),
      in_specs=[
          pl.BlockSpec((br, D), lambda i: (i, 0)),
          pl.BlockSpec((br, 1), lambda i: (i, 0)),
          pl.BlockSpec((br, D), lambda i: (i, 0)),
          pl.BlockSpec((1, D), lambda i: (0, 0)),
          pl.BlockSpec((1, D), lambda i: (0, 0)),
      ],
      out_specs=pl.BlockSpec((br, D), lambda i: (i, 0)),
      out_shape=jax.ShapeDtypeStruct((n, D), f32),
  )(agg, cnt, h_prev, g, b)


def _tc_edge_mm_body(v_ref, g_ref, b_ref, w_ref, bias_ref, m_ref):
  e1 = _ln(v_ref[...], g_ref[...], b_ref[...])
  m_ref[...] = (jnp.dot(e1, w_ref[...], preferred_element_type=f32)
                + bias_ref[...])


def _tc_edge_mm(v, g, b, w, bias):
  n = v.shape[0]
  m = w.shape[1]
  br = 500
  return pl.pallas_call(
      _tc_edge_mm_body,
      grid=(n // br,),
      in_specs=[
          pl.BlockSpec((br, D), lambda i: (i, 0)),
          pl.BlockSpec((1, D), lambda i: (0, 0)),
          pl.BlockSpec((1, D), lambda i: (0, 0)),
          pl.BlockSpec((D, m), lambda i: (0, 0)),
          pl.BlockSpec((1, m), lambda i: (0, 0)),
      ],
      out_specs=pl.BlockSpec((br, m), lambda i: (i, 0)),
      out_shape=jax.ShapeDtypeStruct((n, m), f32),
  )(v, g, b, w, bias)


def _tc_score_body(ssq_ref, q_ref, o_ref):
  o_ref[...] = -jnp.sqrt(ssq_ref[...] + 1e-12) * q_ref[...]


def _tc_score(ssq, q):
  return pl.pallas_call(
      _tc_score_body,
      grid=(1,),
      in_specs=[
          pl.BlockSpec(ssq.shape, lambda i: (0, 0)),
          pl.BlockSpec(q.shape, lambda i: (0, 0)),
      ],
      out_specs=pl.BlockSpec(ssq.shape, lambda i: (0, 0)),
      out_shape=jax.ShapeDtypeStruct(ssq.shape, f32),
  )(ssq, q)


AGG_CH = 128
AGG_FULL = 78
AGG_TAIL = 16
EDGE_CH = 40
EDGE_IT = 125
PER_TILE_EDGES = N_EDGES // (NC * NS)


def _dst_indices(i1_ref, i0_ref, dF_ref, dB_ref, ch, base_node):
  for k in range(ch // L):
    sl = pl.ds(k * L, L)
    d1 = i1_ref[sl] - base_node
    dF_ref[sl] = jnp.where((d1 >= 0) & (d1 < HALF), d1, TRASH)
    d0 = i0_ref[sl] - base_node
    dB_ref[sl] = jnp.where((d0 >= 0) & (d0 < HALF), d0, TRASH)


def _sc_agg0_body(pf, pb, rf, rb, ht0, ht1, rr, zrows, zrows_c, ones_h,
                  agg_out, cnt_out,
                  acc, cacc, i0, i1, ir, dF, dB, rows, ones_v,
                  i0t, i1t, irt, dFt, dBt, rowst, ones_t, sem):
  c = lax.axis_index("c")
  s = lax.axis_index("s")
  base_node = c * HALF

  pltpu.sync_copy(zrows, acc.at[pl.ds(s * RPT, RPT)])
  pltpu.sync_copy(zrows_c, cacc.at[pl.ds(s * RPT, RPT)])
  pltpu.sync_copy(ones_h, ones_v)
  pltpu.sync_copy(ones_h.at[pl.ds(0, AGG_TAIL)], ones_t)
  plsc.subcore_barrier()

  def chunk(base, ch, bi0, bi1, bir, bdF, bdB, brows, bones):
    pltpu.sync_copy(ht0.at[pl.ds(base, ch)], bi0)
    pltpu.sync_copy(ht1.at[pl.ds(base, ch)], bi1)
    pltpu.sync_copy(rr.at[pl.ds(base, ch)], bir)
    _dst_indices(bi1, bi0, bdF, bdB, ch, base_node)
    pltpu.async_copy(pf.at[bi0], brows, sem).wait()
    pltpu.sync_copy(brows, acc.at[bdF], add=True)
    pltpu.async_copy(rf.at[bir], brows, sem).wait()
    pltpu.sync_copy(brows, acc.at[bdF], add=True)
    pltpu.sync_copy(bones, cacc.at[bdF], add=True)
    pltpu.async_copy(pb.at[bi1], brows, sem).wait()
    pltpu.sync_copy(brows, acc.at[bdB], add=True)
    pltpu.async_copy(rb.at[bir], brows, sem).wait()
    pltpu.sync_copy(brows, acc.at[bdB], add=True)
    pltpu.sync_copy(bones, cacc.at[bdB], add=True)

  tile_base = s * (N_EDGES // NS)

  def loop_body(g, carry):
    chunk(tile_base + g * AGG_CH, AGG_CH, i0, i1, ir, dF, dB, rows, ones_v)
    return carry

  lax.fori_loop(0, AGG_FULL, loop_body, 0)
  chunk(tile_base + AGG_FULL * AGG_CH, AGG_TAIL,
        i0t, i1t, irt, dFt, dBt, rowst, ones_t)

  plsc.subcore_barrier()
  pltpu.sync_copy(acc.at[pl.ds(s * RPT, RPT)],
                  agg_out.at[pl.ds(c * HALF_P + s * RPT, RPT)])
  pltpu.sync_copy(cacc.at[pl.ds(s * RPT, RPT)],
                  cnt_out.at[pl.ds(c * HALF_P + s * RPT, RPT)])


_sc_agg0 = functools.partial(
    pl.kernel,
    out_type=[
        jax.ShapeDtypeStruct((NC * HALF_P, D), f32),
        jax.ShapeDtypeStruct((NC * HALF_P, CW), f32),
    ],
    mesh=_mesh,
    scratch_types=[
        pltpu.VMEM_SHARED((HALF_P, D), f32),
        pltpu.VMEM_SHARED((HALF_P, CW), f32),
        pltpu.VMEM((AGG_CH,), i32),
        pltpu.VMEM((AGG_CH,), i32),
        pltpu.VMEM((AGG_CH,), i32),
        pltpu.VMEM((AGG_CH,), i32),
        pltpu.VMEM((AGG_CH,), i32),
        pltpu.VMEM((AGG_CH, D), f32),
        pltpu.VMEM((AGG_CH, CW), f32),
        pltpu.VMEM((AGG_TAIL,), i32),
        pltpu.VMEM((AGG_TAIL,), i32),
        pltpu.VMEM((AGG_TAIL,), i32),
        pltpu.VMEM((AGG_TAIL,), i32),
        pltpu.VMEM((AGG_TAIL,), i32),
        pltpu.VMEM((AGG_TAIL, D), f32),
        pltpu.VMEM((AGG_TAIL, CW), f32),
        pltpu.SemaphoreType.DMA,
    ],
)(_sc_agg0_body)


def _sc_agg1_body(mf, mb, pf, pb, ht0, ht1, rr, zrows,
                  agg_out,
                  acc, i0, i1, dF, dB, rows,
                  i0t, i1t, dFt, dBt, rowst, sem):
  c = lax.axis_index("c")
  s = lax.axis_index("s")
  base_node = c * HALF

  pltpu.sync_copy(zrows, acc.at[pl.ds(s * RPT, RPT)])
  plsc.subcore_barrier()

  def chunk(base, ch, bi0, bi1, bdF, bdB, brows):
    pltpu.sync_copy(ht0.at[pl.ds(base, ch)], bi0)
    pltpu.sync_copy(ht1.at[pl.ds(base, ch)], bi1)
    _dst_indices(bi1, bi0, bdF, bdB, ch, base_node)
    pltpu.sync_copy(mf.at[pl.ds(base, ch)], brows)
    pltpu.sync_copy(brows, acc.at[bdF], add=True)
    pltpu.async_copy(pf.at[bi0], brows, sem).wait()
    pltpu.sync_copy(brows, acc.at[bdF], add=True)
    pltpu.sync_copy(mb.at[pl.ds(base, ch)], brows)
    pltpu.sync_copy(brows, acc.at[bdB], add=True)
    pltpu.async_copy(pb.at[bi1], brows, sem).wait()
    pltpu.sync_copy(brows, acc.at[bdB], add=True)

  tile_base = s * (N_EDGES // NS)

  def loop_body(g, carry):
    chunk(tile_base + g * AGG_CH, AGG_CH, i0, i1, dF, dB, rows)
    return carry

  lax.fori_loop(0, AGG_FULL, loop_body, 0)
  chunk(tile_base + AGG_FULL * AGG_CH, AGG_TAIL, i0t, i1t, dFt, dBt, rowst)

  plsc.subcore_barrier()
  pltpu.sync_copy(acc.at[pl.ds(s * RPT, RPT)],
                  agg_out.at[pl.ds(c * HALF_P + s * RPT, RPT)])


_sc_agg1 = functools.partial(
    pl.kernel,
    out_type=jax.ShapeDtypeStruct((NC * HALF_P, D), f32),
    mesh=_mesh,
    scratch_types=[
        pltpu.VMEM_SHARED((HALF_P, D), f32),
        pltpu.VMEM((AGG_CH,), i32),
        pltpu.VMEM((AGG_CH,), i32),
        pltpu.VMEM((AGG_CH,), i32),
        pltpu.VMEM((AGG_CH,), i32),
        pltpu.VMEM((AGG_CH, D), f32),
        pltpu.VMEM((AGG_TAIL,), i32),
        pltpu.VMEM((AGG_TAIL,), i32),
        pltpu.VMEM((AGG_TAIL,), i32),
        pltpu.VMEM((AGG_TAIL,), i32),
        pltpu.VMEM((AGG_TAIL, D), f32),
        pltpu.SemaphoreType.DMA,
    ],
)(_sc_agg1_body)


def _sc_edge_vec_body(qh, qt, re, rel0, ht0, ht1, rr,
                      v_out,
                      i0, i1, ir, bqh, bqt, bre, br0, sem):
  c = lax.axis_index("c")
  s = lax.axis_index("s")
  wid = s * NC + c
  tile_base = wid * PER_TILE_EDGES

  def loop_body(g, carry):
    base = tile_base + g * EDGE_CH
    pltpu.sync_copy(ht0.at[pl.ds(base, EDGE_CH)], i0)
    pltpu.sync_copy(ht1.at[pl.ds(base, EDGE_CH)], i1)
    pltpu.sync_copy(rr.at[pl.ds(base, EDGE_CH)], ir)
    pltpu.async_copy(qh.at[i0], bqh, sem).wait()
    pltpu.async_copy(qt.at[i1], bqt, sem).wait()
    pltpu.async_copy(re.at[ir], bre, sem).wait()
    pltpu.async_copy(rel0.at[ir], br0, sem).wait()

    def row_body(i, rcarry):
      for j in range(D // L):
        sl = pl.ds(j * L, L)
        q = bqh[i, sl] + bqt[i, sl] + bre[i, sl]
        bqh[i, sl] = jnp.where(q > 0, q, 0.01 * q) + br0[i, sl]
      return rcarry

    lax.fori_loop(0, EDGE_CH, row_body, 0)
    pltpu.sync_copy(bqh, v_out.at[pl.ds(base, EDGE_CH)])
    return carry

  lax.fori_loop(0, EDGE_IT, loop_body, 0)


_sc_edge_vec = functools.partial(
    pl.kernel,
    out_type=jax.ShapeDtypeStruct((N_EDGES, D), f32),
    mesh=_mesh,
    scratch_types=[
        pltpu.VMEM((EDGE_CH,), i32),
        pltpu.VMEM((EDGE_CH,), i32),
        pltpu.VMEM((EDGE_CH,), i32),
        pltpu.VMEM((EDGE_CH, D), f32),
        pltpu.VMEM((EDGE_CH, D), f32),
        pltpu.VMEM((EDGE_CH, D), f32),
        pltpu.VMEM((EDGE_CH, D), f32),
        pltpu.SemaphoreType.DMA,
    ],
)(_sc_edge_vec_body)


def _sc_decoder_body(h2, relv, ht0, ht1, rr,
                     ssq_out,
                     i0, i1, ir, bh, bt, brv, bout, sem):
  c = lax.axis_index("c")
  s = lax.axis_index("s")
  wid = s * NC + c
  tile_base = wid * PER_TILE_EDGES

  def loop_body(g, carry):
    base = tile_base + g * EDGE_CH
    pltpu.sync_copy(ht0.at[pl.ds(base, EDGE_CH)], i0)
    pltpu.sync_copy(ht1.at[pl.ds(base, EDGE_CH)], i1)
    pltpu.sync_copy(rr.at[pl.ds(base, EDGE_CH)], ir)
    pltpu.async_copy(h2.at[i0], bh, sem).wait()
    pltpu.async_copy(h2.at[i1], bt, sem).wait()
    pltpu.async_copy(relv.at[ir], brv, sem).wait()

    def row_body(i, rcarry):
      vacc = jnp.zeros((L,), f32)
      for j in range(D // L):
        sl = pl.ds(j * L, L)
        t = bh[i, sl] + brv[i, sl] - bt[i, sl]
        vacc = vacc + t * t
      bout[i] = jnp.sum(vacc)
      return rcarry

    lax.fori_loop(0, EDGE_CH, row_body, 0)
    pltpu.sync_copy(bout, ssq_out.at[pl.ds(base, EDGE_CH)])
    return carry

  lax.fori_loop(0, EDGE_IT, loop_body, 0)


_sc_decoder = functools.partial(
    pl.kernel,
    out_type=jax.ShapeDtypeStruct((N_EDGES,), f32),
    mesh=_mesh,
    scratch_types=[
        pltpu.VMEM((EDGE_CH,), i32),
        pltpu.VMEM((EDGE_CH,), i32),
        pltpu.VMEM((EDGE_CH,), i32),
        pltpu.VMEM((EDGE_CH, D), f32),
        pltpu.VMEM((EDGE_CH, D), f32),
        pltpu.VMEM((EDGE_CH, D), f32),
        pltpu.VMEM((EDGE_CH,), f32),
        pltpu.SemaphoreType.DMA,
    ],
)(_sc_decoder_body)


def _unpad_half(x):
  return jnp.concatenate([x[:HALF], x[HALF_P:HALF_P + HALF]], axis=0)


def kernel(entity_feat, W_in, b_in, rel_emb, msgF_W, msgF_b, msgB_W, msgB_b,
           mp_g, mp_b, edge_W, edge_b, en_g, en_b, rel_vec, ht, r_tensor,
           queries):
  ht0 = ht[:, 0].astype(i32)
  ht1 = ht[:, 1].astype(i32)
  rr = r_tensor.astype(i32)

  wcat0 = jnp.concatenate([msgF_W[0][:D], msgB_W[0][:D]], axis=1)
  rel_w = jnp.concatenate(
      [msgF_W[0][D:], msgB_W[0][D:], edge_W[0][D:2 * D]], axis=1)
  rel_b = jnp.concatenate([msgF_b[0], msgB_b[0], edge_b[0]])[None]
  rel_pad = jnp.pad(rel_emb, ((0, 12), (0, 0)))
  wcat1 = jnp.concatenate(
      [edge_W[0][:D], edge_W[0][2 * D:], msgF_W[1][:D], msgB_W[1][:D]],
      axis=1)
  w_edge1 = jnp.concatenate([msgF_W[1][D:], msgB_W[1][D:]], axis=1)
  b_edge1 = jnp.concatenate([msgF_b[1], msgB_b[1]])[None]

  zrows = jnp.zeros((RPT, D), f32)
  zrows_c = jnp.zeros((RPT, CW), f32)
  ones_h = jnp.ones((AGG_CH, CW), f32)

  h0, p0 = _tc_in_proj(entity_feat, W_in, b_in[None], wcat0)
  pf0 = p0[:, :D]
  pb0 = p0[:, D:]
  rel_t = _tc_matmul_bias(rel_pad, rel_w, rel_b)
  rf0 = rel_t[:, :D]
  rb0 = rel_t[:, D:2 * D]
  re0 = rel_t[:, 2 * D:]

  agg0_p, cnt_p = _sc_agg0(pf0, pb0, rf0, rb0, ht0, ht1, rr,
                           zrows, zrows_c, ones_h)
  agg0 = _unpad_half(agg0_p)
  cnt = _unpad_half(cnt_p)[:, :1]
  h1, p1 = _tc_node_update(agg0, cnt, h0, mp_g[0][None], mp_b[0][None], wcat1)
  qh = p1[:, :D]
  qt = p1[:, D:2 * D]
  pf1 = p1[:, 2 * D:3 * D]
  pb1 = p1[:, 3 * D:]

  v = _sc_edge_vec(qh, qt, re0, rel_emb, ht0, ht1, rr)

  m = _tc_edge_mm(v, en_g[0][None], en_b[0][None], w_edge1, b_edge1)
  mf = m[:, :D]
  mb = m[:, D:]

  agg1_p = _sc_agg1(mf, mb, pf1, pb1, ht0, ht1, rr, zrows)
  agg1 = _unpad_half(agg1_p)
  h2 = _tc_node_final(agg1, cnt, h1, mp_g[1][None], mp_b[1][None])

  ssq = _sc_decoder(h2, rel_vec, ht0, ht1, rr)
  scores = _tc_score(ssq.reshape(1250, 128),
                     queries.astype(f32).reshape(1250, 128))
  return scores.reshape(N_EDGES)


# speedup vs baseline: 1.1190x; 1.1190x over previous
import functools

import jax
import jax.numpy as jnp
from jax import lax
from jax.experimental import pallas as pl
from jax.experimental.pallas import tpu as pltpu
from jax.experimental.pallas import tpu_sc as plsc

D = 256
N_NODES = 10000
N_EDGES = 160000
NC = 2
NS = 16
L = 16

NT = NC * NS
NOWN = 320
N_NODES_P = NT * NOWN

_mesh = plsc.VectorSubcoreMesh(
    core_axis_name="c", subcore_axis_name="s", num_cores=NC, num_subcores=NS)

f32 = jnp.float32
i32 = jnp.int32


def _leaky(x):
  return jnp.where(x > 0, x, 0.01 * x)


def _ln(x, g, b):
  mu = jnp.mean(x, axis=-1, keepdims=True)
  var = jnp.mean((x - mu) ** 2, axis=-1, keepdims=True)
  return (x - mu) * lax.rsqrt(var + 1e-5) * g + b


def _tc_in_proj_body(x_ref, w_ref, b_ref, wcat_ref, h_ref, p_ref):
  h = _leaky(jnp.dot(x_ref[...], w_ref[...], preferred_element_type=f32)
             + b_ref[...])
  h_ref[...] = h
  p_ref[...] = jnp.dot(h, wcat_ref[...], preferred_element_type=f32)


def _tc_in_proj(x, w, b, wcat):
  n = x.shape[0]
  br = 1000
  return pl.pallas_call(
      _tc_in_proj_body,
      grid=(n // br,),
      in_specs=[
          pl.BlockSpec((br, D), lambda i: (i, 0)),
          pl.BlockSpec((D, D), lambda i: (0, 0)),
          pl.BlockSpec((1, D), lambda i: (0, 0)),
          pl.BlockSpec((D, 2 * D), lambda i: (0, 0)),
      ],
      out_specs=[
          pl.BlockSpec((br, D), lambda i: (i, 0)),
          pl.BlockSpec((br, 2 * D), lambda i: (i, 0)),
      ],
      out_shape=[
          jax.ShapeDtypeStruct((n, D), f32),
          jax.ShapeDtypeStruct((n, 2 * D), f32),
      ],
  )(x, w, b, wcat)


def _tc_matmul_bias_body(x_ref, w_ref, b_ref, o_ref):
  o_ref[...] = (jnp.dot(x_ref[...], w_ref[...], preferred_element_type=f32)
                + b_ref[...])


def _tc_matmul_bias(x, w, b):
  n, k = x.shape
  m = w.shape[1]
  return pl.pallas_call(
      _tc_matmul_bias_body,
      grid=(1,),
      in_specs=[
          pl.BlockSpec((n, k), lambda i: (0, 0)),
          pl.BlockSpec((k, m), lambda i: (0, 0)),
          pl.BlockSpec((1, m), lambda i: (0, 0)),
      ],
      out_specs=pl.BlockSpec((n, m), lambda i: (0, 0)),
      out_shape=jax.ShapeDtypeStruct((n, m), f32),
  )(x, w, b)


def _tc_node_update_body(agg_ref, cnt_ref, hp_ref, g_ref, b_ref, wcat_ref,
                         h_ref, p_ref):
  cnt = jnp.maximum(cnt_ref[...], 1.0)
  h = _ln(_leaky(agg_ref[...] / cnt) + hp_ref[...], g_ref[...], b_ref[...])
  h_ref[...] = h
  p_ref[...] = jnp.dot(h, wcat_ref[...], preferred_element_type=f32)


def _tc_node_update(agg, cnt, h_prev, g, b, wcat):
  n = agg.shape[0]
  m = wcat.shape[1]
  br = 1000
  return pl.pallas_call(
      _tc_node_update_body,
      grid=(n // br,),
      in_specs=[
          pl.BlockSpec((br, D), lambda i: (i, 0)),
          pl.BlockSpec((br, 1), lambda i: (i, 0)),
          pl.BlockSpec((br, D), lambda i: (i, 0)),
          pl.BlockSpec((1, D), lambda i: (0, 0)),
          pl.BlockSpec((1, D), lambda i: (0, 0)),
          pl.BlockSpec((D, m), lambda i: (0, 0)),
      ],
      out_specs=[
          pl.BlockSpec((br, D), lambda i: (i, 0)),
          pl.BlockSpec((br, m), lambda i: (i, 0)),
      ],
      out_shape=[
          jax.ShapeDtypeStruct((n, D), f32),
          jax.ShapeDtypeStruct((n, m), f32),
      ],
  )(agg, cnt, h_prev, g, b, wcat)


def _tc_node_final_body(agg_ref, cnt_ref, hp_ref, g_ref, b_ref, h_ref):
  cnt = jnp.maximum(cnt_ref[...], 1.0)
  h_ref[...] = _ln(_leaky(agg_ref[...] / cnt) + hp_ref[...],
                   g_ref[...], b_ref[...])


def _tc_node_final(agg, cnt, h_prev, g, b):
  n = agg.shape[0]
  br = 1000
  return pl.pallas_call(
      _tc_node_final_body,
      grid=(n // br,),
      in_specs=[
          pl.BlockSpec((br, D), lambda i: (i, 0)),
          pl.BlockSpec((br, 1), lambda i: (i, 0)),
          pl.BlockSpec((br, D), lambda i: (i, 0)),
          pl.BlockSpec((1, D), lambda i: (0, 0)),
          pl.BlockSpec((1, D), lambda i: (0, 0)),
      ],
      out_specs=pl.BlockSpec((br, D), lambda i: (i, 0)),
      out_shape=jax.ShapeDtypeStruct((n, D), f32),
  )(agg, cnt, h_prev, g, b)


def _tc_edge_mm_body(v_ref, g_ref, b_ref, w_ref, bias_ref, m_ref):
  e1 = _ln(v_ref[...], g_ref[...], b_ref[...])
  m_ref[...] = (jnp.dot(e1, w_ref[...], preferred_element_type=f32)
                + bias_ref[...])


def _tc_edge_mm(v, g, b, w, bias):
  n = v.shape[0]
  m = w.shape[1]
  br = 640
  return pl.pallas_call(
      _tc_edge_mm_body,
      grid=(n // br,),
      in_specs=[
          pl.BlockSpec((br, D), lambda i: (i, 0)),
          pl.BlockSpec((1, D), lambda i: (0, 0)),
          pl.BlockSpec((1, D), lambda i: (0, 0)),
          pl.BlockSpec((D, m), lambda i: (0, 0)),
          pl.BlockSpec((1, m), lambda i: (0, 0)),
      ],
      out_specs=pl.BlockSpec((br, m), lambda i: (i, 0)),
      out_shape=jax.ShapeDtypeStruct((n, m), f32),
  )(v, g, b, w, bias)


def _tc_score_body(ssq_ref, q_ref, o_ref):
  o_ref[...] = -jnp.sqrt(ssq_ref[...] + 1e-12) * q_ref[...]


def _tc_score(ssq, q):
  return pl.pallas_call(
      _tc_score_body,
      grid=(1,),
      in_specs=[
          pl.BlockSpec(ssq.shape, lambda i: (0, 0)),
          pl.BlockSpec(q.shape, lambda i: (0, 0)),
      ],
      out_specs=pl.BlockSpec(ssq.shape, lambda i: (0, 0)),
      out_shape=jax.ShapeDtypeStruct(ssq.shape, f32),
  )(ssq, q)


SCAN_CH = 1024
SCAN_FULL = 156
SCAN_TAIL = 256
SB = 64
CBUF = 1104


def _zero_acc(acc, cnt2):
  def zr(i, carry):
    for j in range(D // L):
      acc[i, pl.ds(j * L, L)] = jnp.zeros((L,), f32)
    return carry
  lax.fori_loop(0, NOWN, zr, 0)
  if cnt2 is not None:
    for rI in range(3):
      for k in range(128 // L):
        cnt2[rI, pl.ds(k * L, L)] = jnp.zeros((L,), f32)


def _drain(acc, cnt2, gidx, rows, sem, d_big, s_big, r_big, tbl_s, tbl_r,
           cur, lanes, flush):
  if flush:
    nb = (cur + SB - 1) // SB
  else:
    nb = cur // SB

  def sub(b, carry):
    off = b * SB
    if flush:
      live = jnp.minimum(cur - off, SB)
    else:
      live = SB
    for k in range(SB // L):
      gidx[pl.ds(k * L, L)] = s_big[pl.ds(off + k * L, L)]
    pltpu.async_copy(tbl_s.at[gidx], rows, sem).wait()

    def add1(i, c2):
      dloc = d_big[pl.ds(off + i, L)][0]
      for j in range(D // L):
        sl = pl.ds(j * L, L)
        plsc.addupdate(acc.at[dloc, sl], rows[i, sl])
      return c2

    lax.fori_loop(0, live, add1, 0)
    for k in range(SB // L):
      gidx[pl.ds(k * L, L)] = r_big[pl.ds(off + k * L, L)]
    pltpu.async_copy(tbl_r.at[gidx], rows, sem).wait()

    def add2(i, c2):
      dloc = d_big[pl.ds(off + i, L)][0]
      for j in range(D // L):
        sl = pl.ds(j * L, L)
        plsc.addupdate(acc.at[dloc, sl], rows[i, sl])
      if cnt2 is not None:
        rI = dloc // 128
        lg = (dloc % 128) // L
        ln = dloc % L
        slc = pl.ds(lg * L, L)
        cnt2[rI, slc] = cnt2[rI, slc] + jnp.where(lanes == ln, 1.0, 0.0)
      return c2

    lax.fori_loop(0, live, add2, 0)
    return carry

  lax.fori_loop(0, nb, sub, 0)
  if flush:
    return cur * 0
  lo = nb * SB
  for k in range(SB // L):
    dv = d_big[pl.ds(lo + k * L, L)]
    sv = s_big[pl.ds(lo + k * L, L)]
    rv = r_big[pl.ds(lo + k * L, L)]
    d_big[pl.ds(k * L, L)] = dv
    s_big[pl.ds(k * L, L)] = sv
    r_big[pl.ds(k * L, L)] = rv
  return cur - lo


def _sc_agg_body(with_counts, pfs, pbs, rfs, rbs, ht0, ht1, rr,
                 agg_out, cnt_out, acc, cnt2,
                 i0c, i1c, irc, sF, rF, dF, sB2, rB2, dB2, gidx, rows, sem):
  c = lax.axis_index("c")
  s = lax.axis_index("s")
  w = s * NC + c
  wbase = w * NOWN
  lanes = lax.iota(i32, L)

  _zero_acc(acc, cnt2)

  def zc(i, carry):
    zv = jnp.zeros((L,), i32)
    for buf in (sF, rF, dF, sB2, rB2, dB2):
      buf[pl.ds(i * L, L)] = zv
    return carry
  lax.fori_loop(0, CBUF // L, zc, 0)

  def chunk(base, n, curF, curB):
    pltpu.sync_copy(ht0.at[pl.ds(base, n)], i0c.at[pl.ds(0, n)])
    pltpu.sync_copy(ht1.at[pl.ds(base, n)], i1c.at[pl.ds(0, n)])
    if rr is not None:
      pltpu.sync_copy(rr.at[pl.ds(base, n)], irc.at[pl.ds(0, n)])
    for k in range(n // L):
      sl = pl.ds(k * L, L)
      src0 = i0c[sl]
      src1 = i1c[sl]
      if rr is not None:
        ridx = irc[sl]
      else:
        ridx = base + k * L + lanes
      lv = src1 - wbase
      m = (lv >= 0) & (lv < NOWN)
      keys = jnp.where(m, lanes, 2 * L + lanes)
      _, sd = plsc.sort_key_val(keys, lv)
      _, ss = plsc.sort_key_val(keys, src0)
      _, sr = plsc.sort_key_val(keys, ridx)
      dF[pl.ds(curF, L)] = sd
      sF[pl.ds(curF, L)] = ss
      rF[pl.ds(curF, L)] = sr
      curF = curF + plsc.all_reduce_population_count(m)[0]
      lv2 = src0 - wbase
      m2 = (lv2 >= 0) & (lv2 < NOWN)
      keys2 = jnp.where(m2, lanes, 2 * L + lanes)
      _, sd2 = plsc.sort_key_val(keys2, lv2)
      _, ss2 = plsc.sort_key_val(keys2, src1)
      _, sr2 = plsc.sort_key_val(keys2, ridx)
      dB2[pl.ds(curB, L)] = sd2
      sB2[pl.ds(curB, L)] = ss2
      rB2[pl.ds(curB, L)] = sr2
      curB = curB + plsc.all_reduce_population_count(m2)[0]
    curF = _drain(acc, cnt2, gidx, rows, sem, dF, sF, rF, pfs, rfs,
                  curF, lanes, False)
    curB = _drain(acc, cnt2, gidx, rows, sem, dB2, sB2, rB2, pbs, rbs,
                  curB, lanes, False)
    return curF, curB

  def loop_body(g, carry):
    return chunk(g * SCAN_CH, SCAN_CH, carry[0], carry[1])

  z = jnp.zeros((), i32)
  curF, curB = lax.fori_loop(0, SCAN_FULL, loop_body, (z, z))
  curF, curB = chunk(SCAN_FULL * SCAN_CH, SCAN_TAIL, curF, curB)
  _drain(acc, cnt2, gidx, rows, sem, dF, sF, rF, pfs, rfs, curF, lanes, True)
  _drain(acc, cnt2, gidx, rows, sem, dB2, sB2, rB2, pbs, rbs, curB, lanes,
         True)

  pltpu.sync_copy(acc, agg_out.at[pl.ds(w * NOWN, NOWN)])
  if with_counts:
    pltpu.sync_copy(cnt2, cnt_out.at[w])


def _agg0_body(pfs, pbs, rfs, rbs, ht0, ht1, rr, agg_out, cnt_out,
               acc, cnt2, i0c, i1c, irc, sF, rF, dF, sB2, rB2, dB2,
               gidx, rows, sem):
  _sc_agg_body(True, pfs, pbs, rfs, rbs, ht0, ht1, rr, agg_out, cnt_out,
               acc, cnt2, i0c, i1c, irc, sF, rF, dF, sB2, rB2, dB2,
               gidx, rows, sem)


_sc_agg0 = functools.partial(
    pl.kernel,
    out_type=[
        jax.ShapeDtypeStruct((N_NODES_P, D), f32),
        jax.ShapeDtypeStruct((NT, 3, 128), f32),
    ],
    mesh=_mesh,
    compiler_params=pltpu.CompilerParams(needs_layout_passes=False),
    scratch_types=[
        pltpu.VMEM((NOWN, D), f32),
        pltpu.VMEM((3, 128), f32),
        pltpu.VMEM((SCAN_CH,), i32),
        pltpu.VMEM((SCAN_CH,), i32),
        pltpu.VMEM((SCAN_CH,), i32),
        pltpu.VMEM((CBUF,), i32),
        pltpu.VMEM((CBUF,), i32),
        pltpu.VMEM((CBUF,), i32),
        pltpu.VMEM((CBUF,), i32),
        pltpu.VMEM((CBUF,), i32),
        pltpu.VMEM((CBUF,), i32),
        pltpu.VMEM((SB,), i32),
        pltpu.VMEM((SB, D), f32),
        pltpu.SemaphoreType.DMA,
    ],
)(_agg0_body)


def _agg1_body(pfs, pbs, mfs, mbs, ht0, ht1, agg_out,
               acc, i0c, i1c, irc, sF, rF, dF, sB2, rB2, dB2,
               gidx, rows, sem):
  _sc_agg_body(False, pfs, pbs, mfs, mbs, ht0, ht1, None, agg_out, None,
               acc, None, i0c, i1c, irc, sF, rF, dF, sB2, rB2, dB2,
               gidx, rows, sem)


_sc_agg1 = functools.partial(
    pl.kernel,
    out_type=jax.ShapeDtypeStruct((N_NODES_P, D), f32),
    mesh=_mesh,
    compiler_params=pltpu.CompilerParams(needs_layout_passes=False),
    scratch_types=[
        pltpu.VMEM((NOWN, D), f32),
        pltpu.VMEM((SCAN_CH,), i32),
        pltpu.VMEM((SCAN_CH,), i32),
        pltpu.VMEM((SCAN_CH,), i32),
        pltpu.VMEM((CBUF,), i32),
        pltpu.VMEM((CBUF,), i32),
        pltpu.VMEM((CBUF,), i32),
        pltpu.VMEM((CBUF,), i32),
        pltpu.VMEM((CBUF,), i32),
        pltpu.VMEM((CBUF,), i32),
        pltpu.VMEM((SB,), i32),
        pltpu.VMEM((SB, D), f32),
        pltpu.SemaphoreType.DMA,
    ],
)(_agg1_body)


N_EDGES_P = 160256
PTE = N_EDGES_P // NT
EDGE_CH = 112
EDGE_FULL = 44
EDGE_TAIL = 80


def _sc_edge_vec_body(qh, qt, re, rel0, ht0, ht1, rr,
                      v_out,
                      i0, i1, ir, i0t, i1t, irt, bqh, bqt, bre, br0, sem):
  c = lax.axis_index("c")
  s = lax.axis_index("s")
  wid = s * NC + c
  tile_base = wid * PTE

  def chunk(base, ch, bi0, bi1, bir):
    pltpu.sync_copy(ht0.at[pl.ds(base, ch)], bi0)
    pltpu.sync_copy(ht1.at[pl.ds(base, ch)], bi1)
    pltpu.sync_copy(rr.at[pl.ds(base, ch)], bir)
    pltpu.async_copy(qh.at[bi0], bqh.at[pl.ds(0, ch)], sem).wait()
    pltpu.async_copy(qt.at[bi1], bqt.at[pl.ds(0, ch)], sem).wait()
    pltpu.async_copy(re.at[bir], bre.at[pl.ds(0, ch)], sem).wait()
    pltpu.async_copy(rel0.at[bir], br0.at[pl.ds(0, ch)], sem).wait()

    def row_body(i, rcarry):
      for j in range(D // L):
        sl = pl.ds(j * L, L)
        q = bqh[i, sl] + bqt[i, sl] + bre[i, sl]
        bqh[i, sl] = jnp.where(q > 0, q, 0.01 * q) + br0[i, sl]
      return rcarry

    lax.fori_loop(0, ch, row_body, 0)
    pltpu.sync_copy(bqh.at[pl.ds(0, ch)], v_out.at[pl.ds(base, ch)])

  def loop_body(g, carry):
    chunk(tile_base + g * EDGE_CH, EDGE_CH, i0, i1, ir)
    return carry

  lax.fori_loop(0, EDGE_FULL, loop_body, 0)
  chunk(tile_base + EDGE_FULL * EDGE_CH, EDGE_TAIL, i0t, i1t, irt)


_sc_edge_vec = functools.partial(
    pl.kernel,
    out_type=jax.ShapeDtypeStruct((N_EDGES_P, D), f32),
    mesh=_mesh,
    compiler_params=pltpu.CompilerParams(needs_layout_passes=False),
    scratch_types=[
        pltpu.VMEM((EDGE_CH,), i32),
        pltpu.VMEM((EDGE_CH,), i32),
        pltpu.VMEM((EDGE_CH,), i32),
        pltpu.VMEM((EDGE_TAIL,), i32),
        pltpu.VMEM((EDGE_TAIL,), i32),
        pltpu.VMEM((EDGE_TAIL,), i32),
        pltpu.VMEM((EDGE_CH, D), f32),
        pltpu.VMEM((EDGE_CH, D), f32),
        pltpu.VMEM((EDGE_CH, D), f32),
        pltpu.VMEM((EDGE_CH, D), f32),
        pltpu.SemaphoreType.DMA,
    ],
)(_sc_edge_vec_body)


def _sc_decoder_body(h2, relv, ht0, ht1, rr,
                     ssq_out,
                     i0, i1, ir, i0t, i1t, irt, bh, bt, brv, bout, sem):
  c = lax.axis_index("c")
  s = lax.axis_index("s")
  wid = s * NC + c
  tile_base = wid * PTE
  lanes = lax.iota(i32, L)

  def chunk(base, ch, bi0, bi1, bir):
    pltpu.sync_copy(ht0.at[pl.ds(base, ch)], bi0)
    pltpu.sync_copy(ht1.at[pl.ds(base, ch)], bi1)
    pltpu.sync_copy(rr.at[pl.ds(base, ch)], bir)
    pltpu.async_copy(h2.at[bi0], bh.at[pl.ds(0, ch)], sem).wait()
    pltpu.async_copy(h2.at[bi1], bt.at[pl.ds(0, ch)], sem).wait()
    pltpu.async_copy(relv.at[bir], brv.at[pl.ds(0, ch)], sem).wait()

    def grp_body(g2, gcarry):
      def edge_body(i, outv):
        e = g2 * L + i
        vacc = jnp.zeros((L,), f32)
        for j in range(D // L):
          sl = pl.ds(j * L, L)
          t = bh[e, sl] + brv[e, sl] - bt[e, sl]
          vacc = vacc + t * t
        return jnp.where(lanes == i, jnp.sum(vacc), outv)

      outv = lax.fori_loop(0, L, edge_body, jnp.zeros((L,), f32))
      bout[pl.ds(g2 * L, L)] = outv
      return gcarry

    lax.fori_loop(0, ch // L, grp_body, 0)
    pltpu.sync_copy(bout.at[pl.ds(0, ch)], ssq_out.at[pl.ds(base, ch)])

  def loop_body(g, carry):
    chunk(tile_base + g * EDGE_CH, EDGE_CH, i0, i1, ir)
    return carry

  lax.fori_loop(0, EDGE_FULL, loop_body, 0)
  chunk(tile_base + EDGE_FULL * EDGE_CH, EDGE_TAIL, i0t, i1t, irt)


_sc_decoder = functools.partial(
    pl.kernel,
    out_type=jax.ShapeDtypeStruct((N_EDGES_P,), f32),
    mesh=_mesh,
    compiler_params=pltpu.CompilerParams(needs_layout_passes=False),
    scratch_types=[
        pltpu.VMEM((EDGE_CH,), i32),
        pltpu.VMEM((EDGE_CH,), i32),
        pltpu.VMEM((EDGE_CH,), i32),
        pltpu.VMEM((EDGE_TAIL,), i32),
        pltpu.VMEM((EDGE_TAIL,), i32),
        pltpu.VMEM((EDGE_TAIL,), i32),
        pltpu.VMEM((EDGE_CH, D), f32),
        pltpu.VMEM((EDGE_CH, D), f32),
        pltpu.VMEM((EDGE_CH, D), f32),
        pltpu.VMEM((EDGE_CH,), f32),
        pltpu.SemaphoreType.DMA,
    ],
)(_sc_decoder_body)


def kernel(entity_feat, W_in, b_in, rel_emb, msgF_W, msgF_b, msgB_W, msgB_b,
           mp_g, mp_b, edge_W, edge_b, en_g, en_b, rel_vec, ht, r_tensor,
           queries):
  ht0 = ht[:, 0].astype(i32)
  ht1 = ht[:, 1].astype(i32)
  rr = r_tensor.astype(i32)
  pad = N_EDGES_P - N_EDGES
  ht0p = jnp.pad(ht0, (0, pad))
  ht1p = jnp.pad(ht1, (0, pad))
  rrp = jnp.pad(rr, (0, pad))

  wcat0 = jnp.concatenate([msgF_W[0][:D], msgB_W[0][:D]], axis=1)
  rel_w = jnp.concatenate(
      [msgF_W[0][D:], msgB_W[0][D:], edge_W[0][D:2 * D]], axis=1)
  rel_b = jnp.concatenate([msgF_b[0], msgB_b[0], edge_b[0]])[None]
  rel_pad = jnp.pad(rel_emb, ((0, 12), (0, 0)))
  wcat1 = jnp.concatenate(
      [edge_W[0][:D], edge_W[0][2 * D:], msgF_W[1][:D], msgB_W[1][:D]],
      axis=1)
  w_edge1 = jnp.concatenate([msgF_W[1][D:], msgB_W[1][D:]], axis=1)
  b_edge1 = jnp.concatenate([msgF_b[1], msgB_b[1]])[None]

  h0, p0 = _tc_in_proj(entity_feat, W_in, b_in[None], wcat0)
  pf0 = p0[:, :D]
  pb0 = p0[:, D:]
  rel_t = _tc_matmul_bias(rel_pad, rel_w, rel_b)
  rf0 = rel_t[:, :D]
  rb0 = rel_t[:, D:2 * D]
  re0 = rel_t[:, 2 * D:]

  agg0_p, cnt_p = _sc_agg0(pf0, pb0, rf0, rb0, ht0, ht1, rr)
  agg0 = agg0_p[:N_NODES]
  cnt = cnt_p.reshape(NT, 384)[:, :NOWN].reshape(N_NODES_P)[:N_NODES, None]
  h1, p1 = _tc_node_update(agg0, cnt, h0, mp_g[0][None], mp_b[0][None], wcat1)
  qh = p1[:, :D]
  qt = p1[:, D:2 * D]
  pf1 = p1[:, 2 * D:3 * D]
  pb1 = p1[:, 3 * D:]

  v = _sc_edge_vec(qh, qt, re0, rel_emb, ht0p, ht1p, rrp)[:N_EDGES]

  m = _tc_edge_mm(v, en_g[0][None], en_b[0][None], w_edge1, b_edge1)
  mf = m[:, :D]
  mb = m[:, D:]

  agg1_p = _sc_agg1(pf1, pb1, mf, mb, ht0, ht1)
  agg1 = agg1_p[:N_NODES]
  h2 = _tc_node_final(agg1, cnt, h1, mp_g[1][None], mp_b[1][None])

  ssq = _sc_decoder(h2, rel_vec, ht0p, ht1p, rrp)[:N_EDGES]
  scores = _tc_score(ssq.reshape(1250, 128),
                     queries.astype(f32).reshape(1250, 128))
  return scores.reshape(N_EDGES)


# speedup vs baseline: 1.3096x; 1.1703x over previous
import functools

import jax
import jax.numpy as jnp
from jax import lax
from jax.experimental import pallas as pl
from jax.experimental.pallas import tpu as pltpu
from jax.experimental.pallas import tpu_sc as plsc

D = 256
N_NODES = 10000
N_EDGES = 160000
NC = 2
NS = 16
L = 16

NT = NC * NS
NOWN = 320
N_NODES_P = NT * NOWN

_mesh = plsc.VectorSubcoreMesh(
    core_axis_name="c", subcore_axis_name="s", num_cores=NC, num_subcores=NS)

f32 = jnp.float32
i32 = jnp.int32


def _leaky(x):
  return jnp.where(x > 0, x, 0.01 * x)


def _ln(x, g, b):
  mu = jnp.mean(x, axis=-1, keepdims=True)
  var = jnp.mean((x - mu) ** 2, axis=-1, keepdims=True)
  return (x - mu) * lax.rsqrt(var + 1e-5) * g + b


def _tc_in_proj_body(x_ref, w_ref, b_ref, wcat_ref, h_ref, p_ref):
  h = _leaky(jnp.dot(x_ref[...], w_ref[...], preferred_element_type=f32)
             + b_ref[...])
  h_ref[...] = h
  p_ref[...] = jnp.dot(h, wcat_ref[...], preferred_element_type=f32)


def _tc_in_proj(x, w, b, wcat):
  n = x.shape[0]
  br = 1000
  return pl.pallas_call(
      _tc_in_proj_body,
      grid=(n // br,),
      in_specs=[
          pl.BlockSpec((br, D), lambda i: (i, 0)),
          pl.BlockSpec((D, D), lambda i: (0, 0)),
          pl.BlockSpec((1, D), lambda i: (0, 0)),
          pl.BlockSpec((D, 2 * D), lambda i: (0, 0)),
      ],
      out_specs=[
          pl.BlockSpec((br, D), lambda i: (i, 0)),
          pl.BlockSpec((br, 2 * D), lambda i: (i, 0)),
      ],
      out_shape=[
          jax.ShapeDtypeStruct((n, D), f32),
          jax.ShapeDtypeStruct((n, 2 * D), f32),
      ],
  )(x, w, b, wcat)


def _tc_matmul_bias_body(x_ref, w_ref, b_ref, o_ref):
  o_ref[...] = (jnp.dot(x_ref[...], w_ref[...], preferred_element_type=f32)
                + b_ref[...])


def _tc_matmul_bias(x, w, b):
  n, k = x.shape
  m = w.shape[1]
  return pl.pallas_call(
      _tc_matmul_bias_body,
      grid=(1,),
      in_specs=[
          pl.BlockSpec((n, k), lambda i: (0, 0)),
          pl.BlockSpec((k, m), lambda i: (0, 0)),
          pl.BlockSpec((1, m), lambda i: (0, 0)),
      ],
      out_specs=pl.BlockSpec((n, m), lambda i: (0, 0)),
      out_shape=jax.ShapeDtypeStruct((n, m), f32),
  )(x, w, b)


def _tc_node_update_body(agg_ref, cnt_ref, hp_ref, g_ref, b_ref, wcat_ref,
                         h_ref, p_ref):
  cnt = jnp.maximum(cnt_ref[...], 1.0)
  h = _ln(_leaky(agg_ref[...] / cnt) + hp_ref[...], g_ref[...], b_ref[...])
  h_ref[...] = h
  p_ref[...] = jnp.dot(h, wcat_ref[...], preferred_element_type=f32)


def _tc_node_update(agg, cnt, h_prev, g, b, wcat):
  n = agg.shape[0]
  m = wcat.shape[1]
  br = 1000
  return pl.pallas_call(
      _tc_node_update_body,
      grid=(n // br,),
      in_specs=[
          pl.BlockSpec((br, D), lambda i: (i, 0)),
          pl.BlockSpec((br, 1), lambda i: (i, 0)),
          pl.BlockSpec((br, D), lambda i: (i, 0)),
          pl.BlockSpec((1, D), lambda i: (0, 0)),
          pl.BlockSpec((1, D), lambda i: (0, 0)),
          pl.BlockSpec((D, m), lambda i: (0, 0)),
      ],
      out_specs=[
          pl.BlockSpec((br, D), lambda i: (i, 0)),
          pl.BlockSpec((br, m), lambda i: (i, 0)),
      ],
      out_shape=[
          jax.ShapeDtypeStruct((n, D), f32),
          jax.ShapeDtypeStruct((n, m), f32),
      ],
  )(agg, cnt, h_prev, g, b, wcat)


def _tc_node_final_body(agg_ref, cnt_ref, hp_ref, g_ref, b_ref, h_ref):
  cnt = jnp.maximum(cnt_ref[...], 1.0)
  h_ref[...] = _ln(_leaky(agg_ref[...] / cnt) + hp_ref[...],
                   g_ref[...], b_ref[...])


def _tc_node_final(agg, cnt, h_prev, g, b):
  n = agg.shape[0]
  br = 1000
  return pl.pallas_call(
      _tc_node_final_body,
      grid=(n // br,),
      in_specs=[
          pl.BlockSpec((br, D), lambda i: (i, 0)),
          pl.BlockSpec((br, 1), lambda i: (i, 0)),
          pl.BlockSpec((br, D), lambda i: (i, 0)),
          pl.BlockSpec((1, D), lambda i: (0, 0)),
          pl.BlockSpec((1, D), lambda i: (0, 0)),
      ],
      out_specs=pl.BlockSpec((br, D), lambda i: (i, 0)),
      out_shape=jax.ShapeDtypeStruct((n, D), f32),
  )(agg, cnt, h_prev, g, b)


def _tc_edge_mm_body(v_ref, g_ref, b_ref, w_ref, bias_ref, m_ref):
  e1 = _ln(v_ref[...], g_ref[...], b_ref[...])
  m_ref[...] = (jnp.dot(e1, w_ref[...], preferred_element_type=f32)
                + bias_ref[...])


def _tc_edge_mm(v, g, b, w, bias):
  n = v.shape[0]
  m = w.shape[1]
  br = 640
  return pl.pallas_call(
      _tc_edge_mm_body,
      grid=(n // br,),
      in_specs=[
          pl.BlockSpec((br, D), lambda i: (i, 0)),
          pl.BlockSpec((1, D), lambda i: (0, 0)),
          pl.BlockSpec((1, D), lambda i: (0, 0)),
          pl.BlockSpec((D, m), lambda i: (0, 0)),
          pl.BlockSpec((1, m), lambda i: (0, 0)),
      ],
      out_specs=pl.BlockSpec((br, m), lambda i: (i, 0)),
      out_shape=jax.ShapeDtypeStruct((n, m), f32),
  )(v, g, b, w, bias)


def _tc_score_body(ssq_ref, q_ref, o_ref):
  o_ref[...] = -jnp.sqrt(ssq_ref[...] + 1e-12) * q_ref[...]


def _tc_score(ssq, q):
  return pl.pallas_call(
      _tc_score_body,
      grid=(1,),
      in_specs=[
          pl.BlockSpec(ssq.shape, lambda i: (0, 0)),
          pl.BlockSpec(q.shape, lambda i: (0, 0)),
      ],
      out_specs=pl.BlockSpec(ssq.shape, lambda i: (0, 0)),
      out_shape=jax.ShapeDtypeStruct(ssq.shape, f32),
  )(ssq, q)


SCAN_CH = 1024
SCAN_FULL = 156
SCAN_TAIL = 256
SB = 64
CBUF = 1104


def _zero_acc(acc, cnt2):
  def zr(i, carry):
    for j in range(D // L):
      acc[i, pl.ds(j * L, L)] = jnp.zeros((L,), f32)
    return carry
  lax.fori_loop(0, NOWN, zr, 0)
  if cnt2 is not None:
    for rI in range(3):
      for k in range(128 // L):
        cnt2[rI, pl.ds(k * L, L)] = jnp.zeros((L,), f32)


def _drain(acc, cnt2, gidx, gidx2, rows, rows2, sem, d_big, s_big, r_big,
           tbl_s, tbl_r, cur, lanes, flush):
  if flush:
    nb = (cur + SB - 1) // SB
  else:
    nb = cur // SB

  def sub(b, carry):
    off = b * SB
    if flush:
      live = jnp.minimum(cur - off, SB)
    else:
      live = SB
    for k in range(SB // L):
      gidx[pl.ds(k * L, L)] = s_big[pl.ds(off + k * L, L)]
      gidx2[pl.ds(k * L, L)] = r_big[pl.ds(off + k * L, L)]
    cpa = pltpu.async_copy(tbl_s.at[gidx], rows, sem)
    cpb = pltpu.async_copy(tbl_r.at[gidx2], rows2, sem)
    cpa.wait()
    cpb.wait()

    def add1(i, c2):
      dloc = d_big[pl.ds(off + i, L)][0]
      for j in range(D // L):
        sl = pl.ds(j * L, L)
        plsc.addupdate(acc.at[dloc, sl], rows[i, sl])
        plsc.addupdate(acc.at[dloc, sl], rows2[i, sl])
      if cnt2 is not None:
        rI = dloc // 128
        lg = (dloc % 128) // L
        ln = dloc % L
        slc = pl.ds(lg * L, L)
        cnt2[rI, slc] = cnt2[rI, slc] + jnp.where(lanes == ln, 1.0, 0.0)
      return c2

    lax.fori_loop(0, live, add1, 0)
    return carry

  lax.fori_loop(0, nb, sub, 0)
  if flush:
    return cur * 0
  lo = nb * SB
  for k in range(SB // L):
    dv = d_big[pl.ds(lo + k * L, L)]
    sv = s_big[pl.ds(lo + k * L, L)]
    rv = r_big[pl.ds(lo + k * L, L)]
    d_big[pl.ds(k * L, L)] = dv
    s_big[pl.ds(k * L, L)] = sv
    r_big[pl.ds(k * L, L)] = rv
  return cur - lo


def _sc_agg_body(with_counts, pfs, pbs, rfs, rbs, ht0, ht1, rr,
                 agg_out, cnt_out, acc, cnt2,
                 i0c, i1c, irc, sF, rF, dF, sB2, rB2, dB2, gidx, gidx2,
                 rows, rows2, sem):
  c = lax.axis_index("c")
  s = lax.axis_index("s")
  w = s * NC + c
  wbase = w * NOWN
  lanes = lax.iota(i32, L)

  _zero_acc(acc, cnt2)

  def zc(i, carry):
    zv = jnp.zeros((L,), i32)
    for buf in (sF, rF, dF, sB2, rB2, dB2):
      buf[pl.ds(i * L, L)] = zv
    return carry
  lax.fori_loop(0, CBUF // L, zc, 0)

  def chunk(base, n, curF, curB):
    l0 = pltpu.async_copy(ht0.at[pl.ds(base, n)], i0c.at[pl.ds(0, n)], sem)
    l1 = pltpu.async_copy(ht1.at[pl.ds(base, n)], i1c.at[pl.ds(0, n)], sem)
    if rr is not None:
      l2 = pltpu.async_copy(rr.at[pl.ds(base, n)], irc.at[pl.ds(0, n)], sem)
    l0.wait()
    l1.wait()
    if rr is not None:
      l2.wait()
    for k in range(n // L):
      sl = pl.ds(k * L, L)
      src0 = i0c[sl]
      src1 = i1c[sl]
      if rr is not None:
        ridx = irc[sl]
      else:
        ridx = base + k * L + lanes
      lv = src1 - wbase
      m = (lv >= 0) & (lv < NOWN)
      keys = jnp.where(m, lanes, 2 * L + lanes)
      _, sd = plsc.sort_key_val(keys, lv)
      _, ss = plsc.sort_key_val(keys, src0)
      _, sr = plsc.sort_key_val(keys, ridx)
      dF[pl.ds(curF, L)] = sd
      sF[pl.ds(curF, L)] = ss
      rF[pl.ds(curF, L)] = sr
      curF = curF + plsc.all_reduce_population_count(m)[0]
      lv2 = src0 - wbase
      m2 = (lv2 >= 0) & (lv2 < NOWN)
      keys2 = jnp.where(m2, lanes, 2 * L + lanes)
      _, sd2 = plsc.sort_key_val(keys2, lv2)
      _, ss2 = plsc.sort_key_val(keys2, src1)
      _, sr2 = plsc.sort_key_val(keys2, ridx)
      dB2[pl.ds(curB, L)] = sd2
      sB2[pl.ds(curB, L)] = ss2
      rB2[pl.ds(curB, L)] = sr2
      curB = curB + plsc.all_reduce_population_count(m2)[0]
    curF = _drain(acc, cnt2, gidx, gidx2, rows, rows2, sem, dF, sF, rF,
                  pfs, rfs, curF, lanes, False)
    curB = _drain(acc, cnt2, gidx, gidx2, rows, rows2, sem, dB2, sB2, rB2,
                  pbs, rbs, curB, lanes, False)
    return curF, curB

  def loop_body(g, carry):
    return chunk(g * SCAN_CH, SCAN_CH, carry[0], carry[1])

  z = jnp.zeros((), i32)
  curF, curB = lax.fori_loop(0, SCAN_FULL, loop_body, (z, z))
  curF, curB = chunk(SCAN_FULL * SCAN_CH, SCAN_TAIL, curF, curB)
  _drain(acc, cnt2, gidx, gidx2, rows, rows2, sem, dF, sF, rF, pfs, rfs,
         curF, lanes, True)
  _drain(acc, cnt2, gidx, gidx2, rows, rows2, sem, dB2, sB2, rB2, pbs, rbs,
         curB, lanes, True)

  pltpu.sync_copy(acc, agg_out.at[pl.ds(w * NOWN, NOWN)])
  if with_counts:
    pltpu.sync_copy(cnt2, cnt_out.at[w])


def _agg0_body(pfs, pbs, rfs, rbs, ht0, ht1, rr, agg_out, cnt_out,
               acc, cnt2, i0c, i1c, irc, sF, rF, dF, sB2, rB2, dB2,
               gidx, gidx2, rows, rows2, sem):
  _sc_agg_body(True, pfs, pbs, rfs, rbs, ht0, ht1, rr, agg_out, cnt_out,
               acc, cnt2, i0c, i1c, irc, sF, rF, dF, sB2, rB2, dB2,
               gidx, gidx2, rows, rows2, sem)


_sc_agg0 = functools.partial(
    pl.kernel,
    out_type=[
        jax.ShapeDtypeStruct((N_NODES_P, D), f32),
        jax.ShapeDtypeStruct((NT, 3, 128), f32),
    ],
    mesh=_mesh,
    compiler_params=pltpu.CompilerParams(needs_layout_passes=False),
    scratch_types=[
        pltpu.VMEM((NOWN, D), f32),
        pltpu.VMEM((3, 128), f32),
        pltpu.VMEM((SCAN_CH,), i32),
        pltpu.VMEM((SCAN_CH,), i32),
        pltpu.VMEM((SCAN_CH,), i32),
        pltpu.VMEM((CBUF,), i32),
        pltpu.VMEM((CBUF,), i32),
        pltpu.VMEM((CBUF,), i32),
        pltpu.VMEM((CBUF,), i32),
        pltpu.VMEM((CBUF,), i32),
        pltpu.VMEM((CBUF,), i32),
        pltpu.VMEM((SB,), i32),
        pltpu.VMEM((SB,), i32),
        pltpu.VMEM((SB, D), f32),
        pltpu.VMEM((SB, D), f32),
        pltpu.SemaphoreType.DMA,
    ],
)(_agg0_body)


def _agg1_body(pfs, pbs, mfs, mbs, ht0, ht1, agg_out,
               acc, i0c, i1c, irc, sF, rF, dF, sB2, rB2, dB2,
               gidx, gidx2, rows, rows2, sem):
  _sc_agg_body(False, pfs, pbs, mfs, mbs, ht0, ht1, None, agg_out, None,
               acc, None, i0c, i1c, irc, sF, rF, dF, sB2, rB2, dB2,
               gidx, gidx2, rows, rows2, sem)


_sc_agg1 = functools.partial(
    pl.kernel,
    out_type=jax.ShapeDtypeStruct((N_NODES_P, D), f32),
    mesh=_mesh,
    compiler_params=pltpu.CompilerParams(needs_layout_passes=False),
    scratch_types=[
        pltpu.VMEM((NOWN, D), f32),
        pltpu.VMEM((SCAN_CH,), i32),
        pltpu.VMEM((SCAN_CH,), i32),
        pltpu.VMEM((SCAN_CH,), i32),
        pltpu.VMEM((CBUF,), i32),
        pltpu.VMEM((CBUF,), i32),
        pltpu.VMEM((CBUF,), i32),
        pltpu.VMEM((CBUF,), i32),
        pltpu.VMEM((CBUF,), i32),
        pltpu.VMEM((CBUF,), i32),
        pltpu.VMEM((SB,), i32),
        pltpu.VMEM((SB,), i32),
        pltpu.VMEM((SB, D), f32),
        pltpu.VMEM((SB, D), f32),
        pltpu.SemaphoreType.DMA,
    ],
)(_agg1_body)


N_EDGES_P = 160256
PTE = N_EDGES_P // NT
EDGE_CH = 112
EDGE_FULL = 44
EDGE_TAIL = 80


def _sc_edge_vec_body(qh, qt, re, rel0, ht0, ht1, rr,
                      v_out,
                      i0, i1, ir, i0t, i1t, irt, bqh, bqt, bre, br0, sem):
  c = lax.axis_index("c")
  s = lax.axis_index("s")
  wid = s * NC + c
  tile_base = wid * PTE

  def chunk(base, ch, bi0, bi1, bir):
    l0 = pltpu.async_copy(ht0.at[pl.ds(base, ch)], bi0, sem)
    l1 = pltpu.async_copy(ht1.at[pl.ds(base, ch)], bi1, sem)
    l2 = pltpu.async_copy(rr.at[pl.ds(base, ch)], bir, sem)
    l0.wait()
    l1.wait()
    l2.wait()
    g0 = pltpu.async_copy(qh.at[bi0], bqh.at[pl.ds(0, ch)], sem)
    g1 = pltpu.async_copy(qt.at[bi1], bqt.at[pl.ds(0, ch)], sem)
    g2 = pltpu.async_copy(re.at[bir], bre.at[pl.ds(0, ch)], sem)
    g3 = pltpu.async_copy(rel0.at[bir], br0.at[pl.ds(0, ch)], sem)
    g0.wait()
    g1.wait()
    g2.wait()
    g3.wait()

    def row_body(i, rcarry):
      for j in range(D // L):
        sl = pl.ds(j * L, L)
        q = bqh[i, sl] + bqt[i, sl] + bre[i, sl]
        bqh[i, sl] = jnp.where(q > 0, q, 0.01 * q) + br0[i, sl]
      return rcarry

    lax.fori_loop(0, ch, row_body, 0)
    pltpu.sync_copy(bqh.at[pl.ds(0, ch)], v_out.at[pl.ds(base, ch)])

  def loop_body(g, carry):
    chunk(tile_base + g * EDGE_CH, EDGE_CH, i0, i1, ir)
    return carry

  lax.fori_loop(0, EDGE_FULL, loop_body, 0)
  chunk(tile_base + EDGE_FULL * EDGE_CH, EDGE_TAIL, i0t, i1t, irt)


_sc_edge_vec = functools.partial(
    pl.kernel,
    out_type=jax.ShapeDtypeStruct((N_EDGES_P, D), f32),
    mesh=_mesh,
    compiler_params=pltpu.CompilerParams(needs_layout_passes=False),
    scratch_types=[
        pltpu.VMEM((EDGE_CH,), i32),
        pltpu.VMEM((EDGE_CH,), i32),
        pltpu.VMEM((EDGE_CH,), i32),
        pltpu.VMEM((EDGE_TAIL,), i32),
        pltpu.VMEM((EDGE_TAIL,), i32),
        pltpu.VMEM((EDGE_TAIL,), i32),
        pltpu.VMEM((EDGE_CH, D), f32),
        pltpu.VMEM((EDGE_CH, D), f32),
        pltpu.VMEM((EDGE_CH, D), f32),
        pltpu.VMEM((EDGE_CH, D), f32),
        pltpu.SemaphoreType.DMA,
    ],
)(_sc_edge_vec_body)


def _sc_decoder_body(h2, relv, ht0, ht1, rr,
                     ssq_out,
                     i0, i1, ir, i0t, i1t, irt, bh, bt, brv, bout, sem):
  c = lax.axis_index("c")
  s = lax.axis_index("s")
  wid = s * NC + c
  tile_base = wid * PTE
  lanes = lax.iota(i32, L)

  def chunk(base, ch, bi0, bi1, bir):
    l0 = pltpu.async_copy(ht0.at[pl.ds(base, ch)], bi0, sem)
    l1 = pltpu.async_copy(ht1.at[pl.ds(base, ch)], bi1, sem)
    l2 = pltpu.async_copy(rr.at[pl.ds(base, ch)], bir, sem)
    l0.wait()
    l1.wait()
    l2.wait()
    g0 = pltpu.async_copy(h2.at[bi0], bh.at[pl.ds(0, ch)], sem)
    g1 = pltpu.async_copy(h2.at[bi1], bt.at[pl.ds(0, ch)], sem)
    g2 = pltpu.async_copy(relv.at[bir], brv.at[pl.ds(0, ch)], sem)
    g0.wait()
    g1.wait()
    g2.wait()

    def grp_body(g2, gcarry):
      def edge_body(i, outv):
        e = g2 * L + i
        vacc = jnp.zeros((L,), f32)
        for j in range(D // L):
          sl = pl.ds(j * L, L)
          t = bh[e, sl] + brv[e, sl] - bt[e, sl]
          vacc = vacc + t * t
        return jnp.where(lanes == i, jnp.sum(vacc), outv)

      outv = lax.fori_loop(0, L, edge_body, jnp.zeros((L,), f32))
      bout[pl.ds(g2 * L, L)] = outv
      return gcarry

    lax.fori_loop(0, ch // L, grp_body, 0)
    pltpu.sync_copy(bout.at[pl.ds(0, ch)], ssq_out.at[pl.ds(base, ch)])

  def loop_body(g, carry):
    chunk(tile_base + g * EDGE_CH, EDGE_CH, i0, i1, ir)
    return carry

  lax.fori_loop(0, EDGE_FULL, loop_body, 0)
  chunk(tile_base + EDGE_FULL * EDGE_CH, EDGE_TAIL, i0t, i1t, irt)


_sc_decoder = functools.partial(
    pl.kernel,
    out_type=jax.ShapeDtypeStruct((N_EDGES_P,), f32),
    mesh=_mesh,
    compiler_params=pltpu.CompilerParams(needs_layout_passes=False),
    scratch_types=[
        pltpu.VMEM((EDGE_CH,), i32),
        pltpu.VMEM((EDGE_CH,), i32),
        pltpu.VMEM((EDGE_CH,), i32),
        pltpu.VMEM((EDGE_TAIL,), i32),
        pltpu.VMEM((EDGE_TAIL,), i32),
        pltpu.VMEM((EDGE_TAIL,), i32),
        pltpu.VMEM((EDGE_CH, D), f32),
        pltpu.VMEM((EDGE_CH, D), f32),
        pltpu.VMEM((EDGE_CH, D), f32),
        pltpu.VMEM((EDGE_CH,), f32),
        pltpu.SemaphoreType.DMA,
    ],
)(_sc_decoder_body)


def kernel(entity_feat, W_in, b_in, rel_emb, msgF_W, msgF_b, msgB_W, msgB_b,
           mp_g, mp_b, edge_W, edge_b, en_g, en_b, rel_vec, ht, r_tensor,
           queries):
  ht0 = ht[:, 0].astype(i32)
  ht1 = ht[:, 1].astype(i32)
  rr = r_tensor.astype(i32)
  pad = N_EDGES_P - N_EDGES
  ht0p = jnp.pad(ht0, (0, pad))
  ht1p = jnp.pad(ht1, (0, pad))
  rrp = jnp.pad(rr, (0, pad))

  wcat0 = jnp.concatenate([msgF_W[0][:D], msgB_W[0][:D]], axis=1)
  rel_w = jnp.concatenate(
      [msgF_W[0][D:], msgB_W[0][D:], edge_W[0][D:2 * D]], axis=1)
  rel_b = jnp.concatenate([msgF_b[0], msgB_b[0], edge_b[0]])[None]
  rel_pad = jnp.pad(rel_emb, ((0, 12), (0, 0)))
  wcat1 = jnp.concatenate(
      [edge_W[0][:D], edge_W[0][2 * D:], msgF_W[1][:D], msgB_W[1][:D]],
      axis=1)
  w_edge1 = jnp.concatenate([msgF_W[1][D:], msgB_W[1][D:]], axis=1)
  b_edge1 = jnp.concatenate([msgF_b[1], msgB_b[1]])[None]

  h0, p0 = _tc_in_proj(entity_feat, W_in, b_in[None], wcat0)
  pf0 = p0[:, :D]
  pb0 = p0[:, D:]
  rel_t = _tc_matmul_bias(rel_pad, rel_w, rel_b)
  rf0 = rel_t[:, :D]
  rb0 = rel_t[:, D:2 * D]
  re0 = rel_t[:, 2 * D:]

  agg0_p, cnt_p = _sc_agg0(pf0, pb0, rf0, rb0, ht0, ht1, rr)
  agg0 = agg0_p[:N_NODES]
  cnt = cnt_p.reshape(NT, 384)[:, :NOWN].reshape(N_NODES_P)[:N_NODES, None]
  h1, p1 = _tc_node_update(agg0, cnt, h0, mp_g[0][None], mp_b[0][None], wcat1)
  qh = p1[:, :D]
  qt = p1[:, D:2 * D]
  pf1 = p1[:, 2 * D:3 * D]
  pb1 = p1[:, 3 * D:]

  v = _sc_edge_vec(qh, qt, re0, rel_emb, ht0p, ht1p, rrp)[:N_EDGES]

  m = _tc_edge_mm(v, en_g[0][None], en_b[0][None], w_edge1, b_edge1)
  mf = m[:, :D]
  mb = m[:, D:]

  agg1_p = _sc_agg1(pf1, pb1, mf, mb, ht0, ht1)
  agg1 = agg1_p[:N_NODES]
  h2 = _tc_node_final(agg1, cnt, h1, mp_g[1][None], mp_b[1][None])

  ssq = _sc_decoder(h2, rel_vec, ht0p, ht1p, rrp)[:N_EDGES]
  scores = _tc_score(ssq.reshape(1250, 128),
                     queries.astype(f32).reshape(1250, 128))
  return scores.reshape(N_EDGES)


# speedup vs baseline: 1.3248x; 1.0116x over previous
import functools

import jax
import jax.numpy as jnp
from jax import lax
from jax.experimental import pallas as pl
from jax.experimental.pallas import tpu as pltpu
from jax.experimental.pallas import tpu_sc as plsc

D = 256
N_NODES = 10000
N_EDGES = 160000
NC = 2
NS = 16
L = 16

NT = NC * NS
NOWN = 320
N_NODES_P = NT * NOWN

_mesh = plsc.VectorSubcoreMesh(
    core_axis_name="c", subcore_axis_name="s", num_cores=NC, num_subcores=NS)

f32 = jnp.float32
i32 = jnp.int32


def _leaky(x):
  return jnp.where(x > 0, x, 0.01 * x)


def _ln(x, g, b):
  mu = jnp.mean(x, axis=-1, keepdims=True)
  var = jnp.mean((x - mu) ** 2, axis=-1, keepdims=True)
  return (x - mu) * lax.rsqrt(var + 1e-5) * g + b


def _tc_in_proj_body(x_ref, w_ref, b_ref, wcat_ref, h_ref, p_ref):
  h = _leaky(jnp.dot(x_ref[...], w_ref[...], preferred_element_type=f32)
             + b_ref[...])
  h_ref[...] = h
  p_ref[...] = jnp.dot(h, wcat_ref[...], preferred_element_type=f32)


def _tc_in_proj(x, w, b, wcat):
  n = x.shape[0]
  br = 1000
  return pl.pallas_call(
      _tc_in_proj_body,
      grid=(n // br,),
      in_specs=[
          pl.BlockSpec((br, D), lambda i: (i, 0)),
          pl.BlockSpec((D, D), lambda i: (0, 0)),
          pl.BlockSpec((1, D), lambda i: (0, 0)),
          pl.BlockSpec((D, 2 * D), lambda i: (0, 0)),
      ],
      out_specs=[
          pl.BlockSpec((br, D), lambda i: (i, 0)),
          pl.BlockSpec((br, 2 * D), lambda i: (i, 0)),
      ],
      out_shape=[
          jax.ShapeDtypeStruct((n, D), f32),
          jax.ShapeDtypeStruct((n, 2 * D), f32),
      ],
  )(x, w, b, wcat)


def _tc_matmul_bias_body(x_ref, w_ref, b_ref, o_ref):
  o_ref[...] = (jnp.dot(x_ref[...], w_ref[...], preferred_element_type=f32)
                + b_ref[...])


def _tc_matmul_bias(x, w, b):
  n, k = x.shape
  m = w.shape[1]
  return pl.pallas_call(
      _tc_matmul_bias_body,
      grid=(1,),
      in_specs=[
          pl.BlockSpec((n, k), lambda i: (0, 0)),
          pl.BlockSpec((k, m), lambda i: (0, 0)),
          pl.BlockSpec((1, m), lambda i: (0, 0)),
      ],
      out_specs=pl.BlockSpec((n, m), lambda i: (0, 0)),
      out_shape=jax.ShapeDtypeStruct((n, m), f32),
  )(x, w, b)


def _tc_node_update_body(agg_ref, cnt_ref, hp_ref, g_ref, b_ref, wcat_ref,
                         h_ref, p_ref):
  cnt = jnp.maximum(cnt_ref[...], 1.0)
  h = _ln(_leaky(agg_ref[...] / cnt) + hp_ref[...], g_ref[...], b_ref[...])
  h_ref[...] = h
  p_ref[...] = jnp.dot(h, wcat_ref[...], preferred_element_type=f32)


def _tc_node_update(agg, cnt, h_prev, g, b, wcat):
  n = agg.shape[0]
  m = wcat.shape[1]
  br = 1000
  return pl.pallas_call(
      _tc_node_update_body,
      grid=(n // br,),
      in_specs=[
          pl.BlockSpec((br, D), lambda i: (i, 0)),
          pl.BlockSpec((br, 1), lambda i: (i, 0)),
          pl.BlockSpec((br, D), lambda i: (i, 0)),
          pl.BlockSpec((1, D), lambda i: (0, 0)),
          pl.BlockSpec((1, D), lambda i: (0, 0)),
          pl.BlockSpec((D, m), lambda i: (0, 0)),
      ],
      out_specs=[
          pl.BlockSpec((br, D), lambda i: (i, 0)),
          pl.BlockSpec((br, m), lambda i: (i, 0)),
      ],
      out_shape=[
          jax.ShapeDtypeStruct((n, D), f32),
          jax.ShapeDtypeStruct((n, m), f32),
      ],
  )(agg, cnt, h_prev, g, b, wcat)


def _tc_node_final_body(agg_ref, cnt_ref, hp_ref, g_ref, b_ref, h_ref):
  cnt = jnp.maximum(cnt_ref[...], 1.0)
  h_ref[...] = _ln(_leaky(agg_ref[...] / cnt) + hp_ref[...],
                   g_ref[...], b_ref[...])


def _tc_node_final(agg, cnt, h_prev, g, b):
  n = agg.shape[0]
  br = 1000
  return pl.pallas_call(
      _tc_node_final_body,
      grid=(n // br,),
      in_specs=[
          pl.BlockSpec((br, D), lambda i: (i, 0)),
          pl.BlockSpec((br, 1), lambda i: (i, 0)),
          pl.BlockSpec((br, D), lambda i: (i, 0)),
          pl.BlockSpec((1, D), lambda i: (0, 0)),
          pl.BlockSpec((1, D), lambda i: (0, 0)),
      ],
      out_specs=pl.BlockSpec((br, D), lambda i: (i, 0)),
      out_shape=jax.ShapeDtypeStruct((n, D), f32),
  )(agg, cnt, h_prev, g, b)


def _tc_edge_mm_body(v_ref, g_ref, b_ref, w_ref, bias_ref, m_ref):
  e1 = _ln(v_ref[...], g_ref[...], b_ref[...])
  m_ref[...] = (jnp.dot(e1, w_ref[...], preferred_element_type=f32)
                + bias_ref[...])


def _tc_edge_mm(v, g, b, w, bias):
  n = v.shape[0]
  m = w.shape[1]
  br = 640
  return pl.pallas_call(
      _tc_edge_mm_body,
      grid=(n // br,),
      in_specs=[
          pl.BlockSpec((br, D), lambda i: (i, 0)),
          pl.BlockSpec((1, D), lambda i: (0, 0)),
          pl.BlockSpec((1, D), lambda i: (0, 0)),
          pl.BlockSpec((D, m), lambda i: (0, 0)),
          pl.BlockSpec((1, m), lambda i: (0, 0)),
      ],
      out_specs=pl.BlockSpec((br, m), lambda i: (i, 0)),
      out_shape=jax.ShapeDtypeStruct((n, m), f32),
  )(v, g, b, w, bias)


def _tc_score_body(ssq_ref, q_ref, o_ref):
  o_ref[...] = -jnp.sqrt(ssq_ref[...] + 1e-12) * q_ref[...]


def _tc_score(ssq, q):
  return pl.pallas_call(
      _tc_score_body,
      grid=(1,),
      in_specs=[
          pl.BlockSpec(ssq.shape, lambda i: (0, 0)),
          pl.BlockSpec(q.shape, lambda i: (0, 0)),
      ],
      out_specs=pl.BlockSpec(ssq.shape, lambda i: (0, 0)),
      out_shape=jax.ShapeDtypeStruct(ssq.shape, f32),
  )(ssq, q)


SCAN_CH = 1024
SCAN_FULL = 156
SCAN_TAIL = 256
SB = 64
CBUF = 1104
MASK14 = (1 << 14) - 1
MASK9 = (1 << 9) - 1


def _zero_acc(acc, cnt2):
  def zr(i, carry):
    for j in range(D // L):
      acc[i, pl.ds(j * L, L)] = jnp.zeros((L,), f32)
    return carry
  lax.fori_loop(0, NOWN, zr, 0)
  if cnt2 is not None:
    for rI in range(3):
      for k in range(128 // L):
        cnt2[rI, pl.ds(k * L, L)] = jnp.zeros((L,), f32)


def _srl(x, n):
  return lax.shift_right_logical(x, n)


def _drain(acc, cnt2, gidx, gidx2, rows, rows2, sem, pk_big, e_big,
           tbl_s, tbl_r, cur, lanes, flush):
  if flush:
    nb = (cur + SB - 1) // SB
  else:
    nb = cur // SB

  def sub(b, carry):
    off = b * SB
    if flush:
      live = jnp.minimum(cur - off, SB)
    else:
      live = SB
    for k in range(SB // L):
      pv = pk_big[pl.ds(off + k * L, L)]
      gidx[pl.ds(k * L, L)] = _srl(pv, 9) & MASK14 if e_big is None else \
          _srl(pv, 0) & MASK14
      if e_big is None:
        gidx2[pl.ds(k * L, L)] = pv & MASK9
      else:
        gidx2[pl.ds(k * L, L)] = e_big[pl.ds(off + k * L, L)]
    cpa = pltpu.async_copy(tbl_s.at[gidx], rows, sem)
    cpb = pltpu.async_copy(tbl_r.at[gidx2], rows2, sem)
    cpa.wait()
    cpb.wait()
    shift = 23 if e_big is None else 14

    def add1(i, c2):
      dloc = _srl(pk_big[pl.ds(off + i, L)][0], shift)
      for j in range(D // L):
        sl = pl.ds(j * L, L)
        plsc.addupdate(acc.at[dloc, sl], rows[i, sl])
        plsc.addupdate(acc.at[dloc, sl], rows2[i, sl])
      if cnt2 is not None:
        rI = dloc // 128
        lg = (dloc % 128) // L
        ln = dloc % L
        slc = pl.ds(lg * L, L)
        cnt2[rI, slc] = cnt2[rI, slc] + jnp.where(lanes == ln, 1.0, 0.0)
      return c2

    lax.fori_loop(0, live, add1, 0)
    return carry

  lax.fori_loop(0, nb, sub, 0)
  if flush:
    return cur * 0
  lo = nb * SB
  for k in range(SB // L):
    pv = pk_big[pl.ds(lo + k * L, L)]
    pk_big[pl.ds(k * L, L)] = pv
    if e_big is not None:
      ev = e_big[pl.ds(lo + k * L, L)]
      e_big[pl.ds(k * L, L)] = ev
  return cur - lo


def _sc_agg_body(with_counts, pfs, pbs, rfs, rbs, ht0, ht1, rr,
                 agg_out, cnt_out, acc, cnt2,
                 i0c, i1c, irc, pkF, eF, pkB, eB, gidx, gidx2,
                 rows, rows2, sem):
  c = lax.axis_index("c")
  s = lax.axis_index("s")
  w = s * NC + c
  wbase = w * NOWN
  lanes = lax.iota(i32, L)

  _zero_acc(acc, cnt2)

  def zc(i, carry):
    zv = jnp.zeros((L,), i32)
    for buf in (pkF, pkB, eF, eB):
      if buf is not None:
        buf[pl.ds(i * L, L)] = zv
    return carry
  lax.fori_loop(0, CBUF // L, zc, 0)

  def chunk(base, n, curF, curB):
    l0 = pltpu.async_copy(ht0.at[pl.ds(base, n)], i0c.at[pl.ds(0, n)], sem)
    l1 = pltpu.async_copy(ht1.at[pl.ds(base, n)], i1c.at[pl.ds(0, n)], sem)
    if rr is not None:
      l2 = pltpu.async_copy(rr.at[pl.ds(base, n)], irc.at[pl.ds(0, n)], sem)
    l0.wait()
    l1.wait()
    if rr is not None:
      l2.wait()
    for k in range(n // L):
      sl = pl.ds(k * L, L)
      src0 = i0c[sl]
      src1 = i1c[sl]
      lv = src1 - wbase
      m = (lv >= 0) & (lv < NOWN)
      keys = jnp.where(m, lanes, 2 * L + lanes)
      lv2 = src0 - wbase
      m2 = (lv2 >= 0) & (lv2 < NOWN)
      keys2 = jnp.where(m2, lanes, 2 * L + lanes)
      if rr is not None:
        ridx = irc[sl]
        pk = (lv << 23) | (src0 << 9) | ridx
        pk2 = (lv2 << 23) | (src1 << 9) | ridx
        _, sp = plsc.sort_key_val(keys, pk)
        pkF[pl.ds(curF, L)] = sp
        _, sp2 = plsc.sort_key_val(keys2, pk2)
        pkB[pl.ds(curB, L)] = sp2
      else:
        eidx = base + k * L + lanes
        pk = (lv << 14) | src0
        pk2 = (lv2 << 14) | src1
        _, sp = plsc.sort_key_val(keys, pk)
        pkF[pl.ds(curF, L)] = sp
        _, se = plsc.sort_key_val(keys, eidx)
        eF[pl.ds(curF, L)] = se
        _, sp2 = plsc.sort_key_val(keys2, pk2)
        pkB[pl.ds(curB, L)] = sp2
        _, se2 = plsc.sort_key_val(keys2, eidx)
        eB[pl.ds(curB, L)] = se2
      curF = curF + plsc.all_reduce_population_count(m)[0]
      curB = curB + plsc.all_reduce_population_count(m2)[0]
    curF = _drain(acc, cnt2, gidx, gidx2, rows, rows2, sem, pkF, eF,
                  pfs, rfs, curF, lanes, False)
    curB = _drain(acc, cnt2, gidx, gidx2, rows, rows2, sem, pkB, eB,
                  pbs, rbs, curB, lanes, False)
    return curF, curB

  def loop_body(g, carry):
    return chunk(g * SCAN_CH, SCAN_CH, carry[0], carry[1])

  z = jnp.zeros((), i32)
  curF, curB = lax.fori_loop(0, SCAN_FULL, loop_body, (z, z))
  curF, curB = chunk(SCAN_FULL * SCAN_CH, SCAN_TAIL, curF, curB)
  _drain(acc, cnt2, gidx, gidx2, rows, rows2, sem, pkF, eF, pfs, rfs,
         curF, lanes, True)
  _drain(acc, cnt2, gidx, gidx2, rows, rows2, sem, pkB, eB, pbs, rbs,
         curB, lanes, True)

  pltpu.sync_copy(acc, agg_out.at[pl.ds(w * NOWN, NOWN)])
  if with_counts:
    pltpu.sync_copy(cnt2, cnt_out.at[w])


def _agg0_body(pfs, pbs, rfs, rbs, ht0, ht1, rr, agg_out, cnt_out,
               acc, cnt2, i0c, i1c, irc, pkF, pkB, gidx, gidx2,
               rows, rows2, sem):
  _sc_agg_body(True, pfs, pbs, rfs, rbs, ht0, ht1, rr, agg_out, cnt_out,
               acc, cnt2, i0c, i1c, irc, pkF, None, pkB, None, gidx, gidx2,
               rows, rows2, sem)


_sc_agg0 = functools.partial(
    pl.kernel,
    out_type=[
        jax.ShapeDtypeStruct((N_NODES_P, D), f32),
        jax.ShapeDtypeStruct((NT, 3, 128), f32),
    ],
    mesh=_mesh,
    compiler_params=pltpu.CompilerParams(needs_layout_passes=False),
    scratch_types=[
        pltpu.VMEM((NOWN, D), f32),
        pltpu.VMEM((3, 128), f32),
        pltpu.VMEM((SCAN_CH,), i32),
        pltpu.VMEM((SCAN_CH,), i32),
        pltpu.VMEM((SCAN_CH,), i32),
        pltpu.VMEM((CBUF,), i32),
        pltpu.VMEM((CBUF,), i32),
        pltpu.VMEM((SB,), i32),
        pltpu.VMEM((SB,), i32),
        pltpu.VMEM((SB, D), f32),
        pltpu.VMEM((SB, D), f32),
        pltpu.SemaphoreType.DMA,
    ],
)(_agg0_body)


def _agg1_body(pfs, pbs, mfs, mbs, ht0, ht1, agg_out,
               acc, i0c, i1c, pkF, eF, pkB, eB, gidx, gidx2,
               rows, rows2, sem):
  _sc_agg_body(False, pfs, pbs, mfs, mbs, ht0, ht1, None, agg_out, None,
               acc, None, i0c, i1c, None, pkF, eF, pkB, eB, gidx, gidx2,
               rows, rows2, sem)


_sc_agg1 = functools.partial(
    pl.kernel,
    out_type=jax.ShapeDtypeStruct((N_NODES_P, D), f32),
    mesh=_mesh,
    compiler_params=pltpu.CompilerParams(needs_layout_passes=False),
    scratch_types=[
        pltpu.VMEM((NOWN, D), f32),
        pltpu.VMEM((SCAN_CH,), i32),
        pltpu.VMEM((SCAN_CH,), i32),
        pltpu.VMEM((CBUF,), i32),
        pltpu.VMEM((CBUF,), i32),
        pltpu.VMEM((CBUF,), i32),
        pltpu.VMEM((CBUF,), i32),
        pltpu.VMEM((SB,), i32),
        pltpu.VMEM((SB,), i32),
        pltpu.VMEM((SB, D), f32),
        pltpu.VMEM((SB, D), f32),
        pltpu.SemaphoreType.DMA,
    ],
)(_agg1_body)


N_EDGES_P = 160256
PTE = N_EDGES_P // NT
EDGE_CH = 112
EDGE_FULL = 44
EDGE_TAIL = 80


def _sc_edge_vec_body(qh, qt, re, rel0, ht0, ht1, rr,
                      v_out,
                      i0, i1, ir, i0t, i1t, irt, bqh, bqt, bre, br0, sem):
  c = lax.axis_index("c")
  s = lax.axis_index("s")
  wid = s * NC + c
  tile_base = wid * PTE

  def chunk(base, ch, bi0, bi1, bir):
    l0 = pltpu.async_copy(ht0.at[pl.ds(base, ch)], bi0, sem)
    l1 = pltpu.async_copy(ht1.at[pl.ds(base, ch)], bi1, sem)
    l2 = pltpu.async_copy(rr.at[pl.ds(base, ch)], bir, sem)
    l0.wait()
    l1.wait()
    l2.wait()
    g0 = pltpu.async_copy(qh.at[bi0], bqh.at[pl.ds(0, ch)], sem)
    g1 = pltpu.async_copy(qt.at[bi1], bqt.at[pl.ds(0, ch)], sem)
    g2 = pltpu.async_copy(re.at[bir], bre.at[pl.ds(0, ch)], sem)
    g3 = pltpu.async_copy(rel0.at[bir], br0.at[pl.ds(0, ch)], sem)
    g0.wait()
    g1.wait()
    g2.wait()
    g3.wait()

    def row_body(i, rcarry):
      for j in range(D // L):
        sl = pl.ds(j * L, L)
        q = bqh[i, sl] + bqt[i, sl] + bre[i, sl]
        bqh[i, sl] = jnp.where(q > 0, q, 0.01 * q) + br0[i, sl]
      return rcarry

    lax.fori_loop(0, ch, row_body, 0)
    pltpu.sync_copy(bqh.at[pl.ds(0, ch)], v_out.at[pl.ds(base, ch)])

  def loop_body(g, carry):
    chunk(tile_base + g * EDGE_CH, EDGE_CH, i0, i1, ir)
    return carry

  lax.fori_loop(0, EDGE_FULL, loop_body, 0)
  chunk(tile_base + EDGE_FULL * EDGE_CH, EDGE_TAIL, i0t, i1t, irt)


_sc_edge_vec = functools.partial(
    pl.kernel,
    out_type=jax.ShapeDtypeStruct((N_EDGES_P, D), f32),
    mesh=_mesh,
    compiler_params=pltpu.CompilerParams(needs_layout_passes=False),
    scratch_types=[
        pltpu.VMEM((EDGE_CH,), i32),
        pltpu.VMEM((EDGE_CH,), i32),
        pltpu.VMEM((EDGE_CH,), i32),
        pltpu.VMEM((EDGE_TAIL,), i32),
        pltpu.VMEM((EDGE_TAIL,), i32),
        pltpu.VMEM((EDGE_TAIL,), i32),
        pltpu.VMEM((EDGE_CH, D), f32),
        pltpu.VMEM((EDGE_CH, D), f32),
        pltpu.VMEM((EDGE_CH, D), f32),
        pltpu.VMEM((EDGE_CH, D), f32),
        pltpu.SemaphoreType.DMA,
    ],
)(_sc_edge_vec_body)


def _sc_decoder_body(h2, relv, ht0, ht1, rr,
                     ssq_out,
                     i0, i1, ir, i0t, i1t, irt, bh, bt, brv, bout, sem):
  c = lax.axis_index("c")
  s = lax.axis_index("s")
  wid = s * NC + c
  tile_base = wid * PTE
  lanes = lax.iota(i32, L)

  def chunk(base, ch, bi0, bi1, bir):
    l0 = pltpu.async_copy(ht0.at[pl.ds(base, ch)], bi0, sem)
    l1 = pltpu.async_copy(ht1.at[pl.ds(base, ch)], bi1, sem)
    l2 = pltpu.async_copy(rr.at[pl.ds(base, ch)], bir, sem)
    l0.wait()
    l1.wait()
    l2.wait()
    g0 = pltpu.async_copy(h2.at[bi0], bh.at[pl.ds(0, ch)], sem)
    g1 = pltpu.async_copy(h2.at[bi1], bt.at[pl.ds(0, ch)], sem)
    g2 = pltpu.async_copy(relv.at[bir], brv.at[pl.ds(0, ch)], sem)
    g0.wait()
    g1.wait()
    g2.wait()

    def grp_body(g2, gcarry):
      def edge_body(i, outv):
        e = g2 * L + i
        vacc = jnp.zeros((L,), f32)
        for j in range(D // L):
          sl = pl.ds(j * L, L)
          t = bh[e, sl] + brv[e, sl] - bt[e, sl]
          vacc = vacc + t * t
        return jnp.where(lanes == i, jnp.sum(vacc), outv)

      outv = lax.fori_loop(0, L, edge_body, jnp.zeros((L,), f32))
      bout[pl.ds(g2 * L, L)] = outv
      return gcarry

    lax.fori_loop(0, ch // L, grp_body, 0)
    pltpu.sync_copy(bout.at[pl.ds(0, ch)], ssq_out.at[pl.ds(base, ch)])

  def loop_body(g, carry):
    chunk(tile_base + g * EDGE_CH, EDGE_CH, i0, i1, ir)
    return carry

  lax.fori_loop(0, EDGE_FULL, loop_body, 0)
  chunk(tile_base + EDGE_FULL * EDGE_CH, EDGE_TAIL, i0t, i1t, irt)


_sc_decoder = functools.partial(
    pl.kernel,
    out_type=jax.ShapeDtypeStruct((N_EDGES_P,), f32),
    mesh=_mesh,
    compiler_params=pltpu.CompilerParams(needs_layout_passes=False),
    scratch_types=[
        pltpu.VMEM((EDGE_CH,), i32),
        pltpu.VMEM((EDGE_CH,), i32),
        pltpu.VMEM((EDGE_CH,), i32),
        pltpu.VMEM((EDGE_TAIL,), i32),
        pltpu.VMEM((EDGE_TAIL,), i32),
        pltpu.VMEM((EDGE_TAIL,), i32),
        pltpu.VMEM((EDGE_CH, D), f32),
        pltpu.VMEM((EDGE_CH, D), f32),
        pltpu.VMEM((EDGE_CH, D), f32),
        pltpu.VMEM((EDGE_CH,), f32),
        pltpu.SemaphoreType.DMA,
    ],
)(_sc_decoder_body)


def kernel(entity_feat, W_in, b_in, rel_emb, msgF_W, msgF_b, msgB_W, msgB_b,
           mp_g, mp_b, edge_W, edge_b, en_g, en_b, rel_vec, ht, r_tensor,
           queries):
  ht0 = ht[:, 0].astype(i32)
  ht1 = ht[:, 1].astype(i32)
  rr = r_tensor.astype(i32)
  pad = N_EDGES_P - N_EDGES
  ht0p = jnp.pad(ht0, (0, pad))
  ht1p = jnp.pad(ht1, (0, pad))
  rrp = jnp.pad(rr, (0, pad))

  wcat0 = jnp.concatenate([msgF_W[0][:D], msgB_W[0][:D]], axis=1)
  rel_w = jnp.concatenate(
      [msgF_W[0][D:], msgB_W[0][D:], edge_W[0][D:2 * D]], axis=1)
  rel_b = jnp.concatenate([msgF_b[0], msgB_b[0], edge_b[0]])[None]
  rel_pad = jnp.pad(rel_emb, ((0, 12), (0, 0)))
  wcat1 = jnp.concatenate(
      [edge_W[0][:D], edge_W[0][2 * D:], msgF_W[1][:D], msgB_W[1][:D]],
      axis=1)
  w_edge1 = jnp.concatenate([msgF_W[1][D:], msgB_W[1][D:]], axis=1)
  b_edge1 = jnp.concatenate([msgF_b[1], msgB_b[1]])[None]

  h0, p0 = _tc_in_proj(entity_feat, W_in, b_in[None], wcat0)
  pf0 = p0[:, :D]
  pb0 = p0[:, D:]
  rel_t = _tc_matmul_bias(rel_pad, rel_w, rel_b)
  rf0 = rel_t[:, :D]
  rb0 = rel_t[:, D:2 * D]
  re0 = rel_t[:, 2 * D:]

  agg0_p, cnt_p = _sc_agg0(pf0, pb0, rf0, rb0, ht0, ht1, rr)
  agg0 = agg0_p[:N_NODES]
  cnt = cnt_p.reshape(NT, 384)[:, :NOWN].reshape(N_NODES_P)[:N_NODES, None]
  h1, p1 = _tc_node_update(agg0, cnt, h0, mp_g[0][None], mp_b[0][None], wcat1)
  qh = p1[:, :D]
  qt = p1[:, D:2 * D]
  pf1 = p1[:, 2 * D:3 * D]
  pb1 = p1[:, 3 * D:]

  v = _sc_edge_vec(qh, qt, re0, rel_emb, ht0p, ht1p, rrp)[:N_EDGES]

  m = _tc_edge_mm(v, en_g[0][None], en_b[0][None], w_edge1, b_edge1)
  mf = m[:, :D]
  mb = m[:, D:]

  agg1_p = _sc_agg1(pf1, pb1, mf, mb, ht0, ht1)
  agg1 = agg1_p[:N_NODES]
  h2 = _tc_node_final(agg1, cnt, h1, mp_g[1][None], mp_b[1][None])

  ssq = _sc_decoder(h2, rel_vec, ht0p, ht1p, rrp)[:N_EDGES]
  scores = _tc_score(ssq.reshape(1250, 128),
                     queries.astype(f32).reshape(1250, 128))
  return scores.reshape(N_EDGES)


# speedup vs baseline: 1.3261x; 1.0010x over previous
import functools

import jax
import jax.numpy as jnp
from jax import lax
from jax.experimental import pallas as pl
from jax.experimental.pallas import tpu as pltpu
from jax.experimental.pallas import tpu_sc as plsc

D = 256
N_NODES = 10000
N_EDGES = 160000
NC = 2
NS = 16
L = 16

NT = NC * NS
NOWN = 320
N_NODES_P = NT * NOWN

_mesh = plsc.VectorSubcoreMesh(
    core_axis_name="c", subcore_axis_name="s", num_cores=NC, num_subcores=NS)

f32 = jnp.float32
i32 = jnp.int32


def _leaky(x):
  return jnp.where(x > 0, x, 0.01 * x)


def _ln(x, g, b):
  mu = jnp.mean(x, axis=-1, keepdims=True)
  var = jnp.mean((x - mu) ** 2, axis=-1, keepdims=True)
  return (x - mu) * lax.rsqrt(var + 1e-5) * g + b


def _tc_in_proj_body(x_ref, w_ref, b_ref, wcat_ref, h_ref, p_ref):
  h = _leaky(jnp.dot(x_ref[...], w_ref[...], preferred_element_type=f32)
             + b_ref[...])
  h_ref[...] = h
  p_ref[...] = jnp.dot(h, wcat_ref[...], preferred_element_type=f32)


def _tc_in_proj(x, w, b, wcat):
  n = x.shape[0]
  br = 1000
  return pl.pallas_call(
      _tc_in_proj_body,
      grid=(n // br,),
      in_specs=[
          pl.BlockSpec((br, D), lambda i: (i, 0)),
          pl.BlockSpec((D, D), lambda i: (0, 0)),
          pl.BlockSpec((1, D), lambda i: (0, 0)),
          pl.BlockSpec((D, 2 * D), lambda i: (0, 0)),
      ],
      out_specs=[
          pl.BlockSpec((br, D), lambda i: (i, 0)),
          pl.BlockSpec((br, 2 * D), lambda i: (i, 0)),
      ],
      out_shape=[
          jax.ShapeDtypeStruct((n, D), f32),
          jax.ShapeDtypeStruct((n, 2 * D), f32),
      ],
  )(x, w, b, wcat)


def _tc_matmul_bias_body(x_ref, w_ref, b_ref, o_ref):
  o_ref[...] = (jnp.dot(x_ref[...], w_ref[...], preferred_element_type=f32)
                + b_ref[...])


def _tc_matmul_bias(x, w, b):
  n, k = x.shape
  m = w.shape[1]
  return pl.pallas_call(
      _tc_matmul_bias_body,
      grid=(1,),
      in_specs=[
          pl.BlockSpec((n, k), lambda i: (0, 0)),
          pl.BlockSpec((k, m), lambda i: (0, 0)),
          pl.BlockSpec((1, m), lambda i: (0, 0)),
      ],
      out_specs=pl.BlockSpec((n, m), lambda i: (0, 0)),
      out_shape=jax.ShapeDtypeStruct((n, m), f32),
  )(x, w, b)


def _tc_node_update_body(agg_ref, cnt_ref, hp_ref, g_ref, b_ref, wcat_ref,
                         h_ref, p_ref):
  cnt = jnp.maximum(cnt_ref[...], 1.0)
  h = _ln(_leaky(agg_ref[...] / cnt) + hp_ref[...], g_ref[...], b_ref[...])
  h_ref[...] = h
  p_ref[...] = jnp.dot(h, wcat_ref[...], preferred_element_type=f32)


def _tc_node_update(agg, cnt, h_prev, g, b, wcat):
  n = agg.shape[0]
  m = wcat.shape[1]
  br = 1000
  return pl.pallas_call(
      _tc_node_update_body,
      grid=(n // br,),
      in_specs=[
          pl.BlockSpec((br, D), lambda i: (i, 0)),
          pl.BlockSpec((br, 1), lambda i: (i, 0)),
          pl.BlockSpec((br, D), lambda i: (i, 0)),
          pl.BlockSpec((1, D), lambda i: (0, 0)),
          pl.BlockSpec((1, D), lambda i: (0, 0)),
          pl.BlockSpec((D, m), lambda i: (0, 0)),
      ],
      out_specs=[
          pl.BlockSpec((br, D), lambda i: (i, 0)),
          pl.BlockSpec((br, m), lambda i: (i, 0)),
      ],
      out_shape=[
          jax.ShapeDtypeStruct((n, D), f32),
          jax.ShapeDtypeStruct((n, m), f32),
      ],
  )(agg, cnt, h_prev, g, b, wcat)


def _tc_node_final_body(agg_ref, cnt_ref, hp_ref, g_ref, b_ref, h_ref):
  cnt = jnp.maximum(cnt_ref[...], 1.0)
  h_ref[...] = _ln(_leaky(agg_ref[...] / cnt) + hp_ref[...],
                   g_ref[...], b_ref[...])


def _tc_node_final(agg, cnt, h_prev, g, b):
  n = agg.shape[0]
  br = 1000
  return pl.pallas_call(
      _tc_node_final_body,
      grid=(n // br,),
      in_specs=[
          pl.BlockSpec((br, D), lambda i: (i, 0)),
          pl.BlockSpec((br, 1), lambda i: (i, 0)),
          pl.BlockSpec((br, D), lambda i: (i, 0)),
          pl.BlockSpec((1, D), lambda i: (0, 0)),
          pl.BlockSpec((1, D), lambda i: (0, 0)),
      ],
      out_specs=pl.BlockSpec((br, D), lambda i: (i, 0)),
      out_shape=jax.ShapeDtypeStruct((n, D), f32),
  )(agg, cnt, h_prev, g, b)


def _tc_edge_mm_body(v_ref, g_ref, b_ref, w_ref, bias_ref, m_ref):
  e1 = _ln(v_ref[...], g_ref[...], b_ref[...])
  m_ref[...] = (jnp.dot(e1, w_ref[...], preferred_element_type=f32)
                + bias_ref[...])


def _tc_edge_mm(v, g, b, w, bias):
  n = v.shape[0]
  m = w.shape[1]
  br = 640
  return pl.pallas_call(
      _tc_edge_mm_body,
      grid=(n // br,),
      in_specs=[
          pl.BlockSpec((br, D), lambda i: (i, 0)),
          pl.BlockSpec((1, D), lambda i: (0, 0)),
          pl.BlockSpec((1, D), lambda i: (0, 0)),
          pl.BlockSpec((D, m), lambda i: (0, 0)),
          pl.BlockSpec((1, m), lambda i: (0, 0)),
      ],
      out_specs=pl.BlockSpec((br, m), lambda i: (i, 0)),
      out_shape=jax.ShapeDtypeStruct((n, m), f32),
  )(v, g, b, w, bias)


def _tc_score_body(ssq_ref, q_ref, o_ref):
  o_ref[...] = -jnp.sqrt(ssq_ref[...] + 1e-12) * q_ref[...]


def _tc_score(ssq, q):
  return pl.pallas_call(
      _tc_score_body,
      grid=(1,),
      in_specs=[
          pl.BlockSpec(ssq.shape, lambda i: (0, 0)),
          pl.BlockSpec(q.shape, lambda i: (0, 0)),
      ],
      out_specs=pl.BlockSpec(ssq.shape, lambda i: (0, 0)),
      out_shape=jax.ShapeDtypeStruct(ssq.shape, f32),
  )(ssq, q)


SCAN_CH = 2048
SCAN_FULL = 78
SCAN_TAIL = 256
SB = 64
CBUF = 2144
MASK14 = (1 << 14) - 1
MASK9 = (1 << 9) - 1


def _zero_acc(acc, cnt2):
  def zr(i, carry):
    for j in range(D // L):
      acc[i, pl.ds(j * L, L)] = jnp.zeros((L,), f32)
    return carry
  lax.fori_loop(0, NOWN, zr, 0)
  if cnt2 is not None:
    for rI in range(3):
      for k in range(128 // L):
        cnt2[rI, pl.ds(k * L, L)] = jnp.zeros((L,), f32)


def _srl(x, n):
  return lax.shift_right_logical(x, n)


def _drain(acc, cnt2, gidx, gidx2, rows, rows2, sem, pk_big, e_big,
           tbl_s, tbl_r, cur, lanes, flush):
  if flush:
    nb = (cur + SB - 1) // SB
  else:
    nb = cur // SB

  def sub(b, carry):
    off = b * SB
    if flush:
      live = jnp.minimum(cur - off, SB)
    else:
      live = SB
    for k in range(SB // L):
      pv = pk_big[pl.ds(off + k * L, L)]
      gidx[pl.ds(k * L, L)] = _srl(pv, 9) & MASK14 if e_big is None else \
          _srl(pv, 0) & MASK14
      if e_big is None:
        gidx2[pl.ds(k * L, L)] = pv & MASK9
      else:
        gidx2[pl.ds(k * L, L)] = e_big[pl.ds(off + k * L, L)]
    cpa = pltpu.async_copy(tbl_s.at[gidx], rows, sem)
    cpb = pltpu.async_copy(tbl_r.at[gidx2], rows2, sem)
    shift = 23 if e_big is None else 14
    cpa.wait()

    def add1(i, c2):
      dloc = _srl(pk_big[pl.ds(off + i, L)][0], shift)
      for j in range(D // L):
        sl = pl.ds(j * L, L)
        plsc.addupdate(acc.at[dloc, sl], rows[i, sl])
      if cnt2 is not None:
        rI = dloc // 128
        lg = (dloc % 128) // L
        ln = dloc % L
        slc = pl.ds(lg * L, L)
        cnt2[rI, slc] = cnt2[rI, slc] + jnp.where(lanes == ln, 1.0, 0.0)
      return c2

    lax.fori_loop(0, live, add1, 0)
    cpb.wait()

    def add2(i, c2):
      dloc = _srl(pk_big[pl.ds(off + i, L)][0], shift)
      for j in range(D // L):
        sl = pl.ds(j * L, L)
        plsc.addupdate(acc.at[dloc, sl], rows2[i, sl])
      return c2

    lax.fori_loop(0, live, add2, 0)
    return carry

  lax.fori_loop(0, nb, sub, 0)
  if flush:
    return cur * 0
  lo = nb * SB
  for k in range(SB // L):
    pv = pk_big[pl.ds(lo + k * L, L)]
    pk_big[pl.ds(k * L, L)] = pv
    if e_big is not None:
      ev = e_big[pl.ds(lo + k * L, L)]
      e_big[pl.ds(k * L, L)] = ev
  return cur - lo


def _sc_agg_body(with_counts, pfs, pbs, rfs, rbs, ht0, ht1, rr,
                 agg_out, cnt_out, acc, cnt2,
                 i0c, i1c, irc, pkF, eF, pkB, eB, gidx, gidx2,
                 rows, rows2, sem):
  c = lax.axis_index("c")
  s = lax.axis_index("s")
  w = s * NC + c
  wbase = w * NOWN
  lanes = lax.iota(i32, L)

  _zero_acc(acc, cnt2)

  def zc(i, carry):
    zv = jnp.zeros((L,), i32)
    for buf in (pkF, pkB, eF, eB):
      if buf is not None:
        buf[pl.ds(i * L, L)] = zv
    return carry
  lax.fori_loop(0, CBUF // L, zc, 0)

  def chunk(base, n, curF, curB):
    l0 = pltpu.async_copy(ht0.at[pl.ds(base, n)], i0c.at[pl.ds(0, n)], sem)
    l1 = pltpu.async_copy(ht1.at[pl.ds(base, n)], i1c.at[pl.ds(0, n)], sem)
    if rr is not None:
      l2 = pltpu.async_copy(rr.at[pl.ds(base, n)], irc.at[pl.ds(0, n)], sem)
    l0.wait()
    l1.wait()
    if rr is not None:
      l2.wait()
    for k in range(n // L):
      sl = pl.ds(k * L, L)
      src0 = i0c[sl]
      src1 = i1c[sl]
      lv = src1 - wbase
      m = (lv >= 0) & (lv < NOWN)
      keys = jnp.where(m, lanes, 2 * L + lanes)
      lv2 = src0 - wbase
      m2 = (lv2 >= 0) & (lv2 < NOWN)
      keys2 = jnp.where(m2, lanes, 2 * L + lanes)
      if rr is not None:
        ridx = irc[sl]
        pk = (lv << 23) | (src0 << 9) | ridx
        pk2 = (lv2 << 23) | (src1 << 9) | ridx
        _, sp = plsc.sort_key_val(keys, pk)
        pkF[pl.ds(curF, L)] = sp
        _, sp2 = plsc.sort_key_val(keys2, pk2)
        pkB[pl.ds(curB, L)] = sp2
      else:
        eidx = base + k * L + lanes
        pk = (lv << 14) | src0
        pk2 = (lv2 << 14) | src1
        _, sp = plsc.sort_key_val(keys, pk)
        pkF[pl.ds(curF, L)] = sp
        _, se = plsc.sort_key_val(keys, eidx)
        eF[pl.ds(curF, L)] = se
        _, sp2 = plsc.sort_key_val(keys2, pk2)
        pkB[pl.ds(curB, L)] = sp2
        _, se2 = plsc.sort_key_val(keys2, eidx)
        eB[pl.ds(curB, L)] = se2
      curF = curF + plsc.all_reduce_population_count(m)[0]
      curB = curB + plsc.all_reduce_population_count(m2)[0]
    curF = _drain(acc, cnt2, gidx, gidx2, rows, rows2, sem, pkF, eF,
                  pfs, rfs, curF, lanes, False)
    curB = _drain(acc, cnt2, gidx, gidx2, rows, rows2, sem, pkB, eB,
                  pbs, rbs, curB, lanes, False)
    return curF, curB

  def loop_body(g, carry):
    return chunk(g * SCAN_CH, SCAN_CH, carry[0], carry[1])

  z = jnp.zeros((), i32)
  curF, curB = lax.fori_loop(0, SCAN_FULL, loop_body, (z, z))
  curF, curB = chunk(SCAN_FULL * SCAN_CH, SCAN_TAIL, curF, curB)
  _drain(acc, cnt2, gidx, gidx2, rows, rows2, sem, pkF, eF, pfs, rfs,
         curF, lanes, True)
  _drain(acc, cnt2, gidx, gidx2, rows, rows2, sem, pkB, eB, pbs, rbs,
         curB, lanes, True)

  pltpu.sync_copy(acc, agg_out.at[pl.ds(w * NOWN, NOWN)])
  if with_counts:
    pltpu.sync_copy(cnt2, cnt_out.at[w])


def _agg0_body(pfs, pbs, rfs, rbs, ht0, ht1, rr, agg_out, cnt_out,
               acc, cnt2, i0c, i1c, irc, pkF, pkB, gidx, gidx2,
               rows, rows2, sem):
  _sc_agg_body(True, pfs, pbs, rfs, rbs, ht0, ht1, rr, agg_out, cnt_out,
               acc, cnt2, i0c, i1c, irc, pkF, None, pkB, None, gidx, gidx2,
               rows, rows2, sem)


_sc_agg0 = functools.partial(
    pl.kernel,
    out_type=[
        jax.ShapeDtypeStruct((N_NODES_P, D), f32),
        jax.ShapeDtypeStruct((NT, 3, 128), f32),
    ],
    mesh=_mesh,
    compiler_params=pltpu.CompilerParams(needs_layout_passes=False),
    scratch_types=[
        pltpu.VMEM((NOWN, D), f32),
        pltpu.VMEM((3, 128), f32),
        pltpu.VMEM((SCAN_CH,), i32),
        pltpu.VMEM((SCAN_CH,), i32),
        pltpu.VMEM((SCAN_CH,), i32),
        pltpu.VMEM((CBUF,), i32),
        pltpu.VMEM((CBUF,), i32),
        pltpu.VMEM((SB,), i32),
        pltpu.VMEM((SB,), i32),
        pltpu.VMEM((SB, D), f32),
        pltpu.VMEM((SB, D), f32),
        pltpu.SemaphoreType.DMA,
    ],
)(_agg0_body)


def _agg1_body(pfs, pbs, mfs, mbs, ht0, ht1, agg_out,
               acc, i0c, i1c, pkF, eF, pkB, eB, gidx, gidx2,
               rows, rows2, sem):
  _sc_agg_body(False, pfs, pbs, mfs, mbs, ht0, ht1, None, agg_out, None,
               acc, None, i0c, i1c, None, pkF, eF, pkB, eB, gidx, gidx2,
               rows, rows2, sem)


_sc_agg1 = functools.partial(
    pl.kernel,
    out_type=jax.ShapeDtypeStruct((N_NODES_P, D), f32),
    mesh=_mesh,
    compiler_params=pltpu.CompilerParams(needs_layout_passes=False),
    scratch_types=[
        pltpu.VMEM((NOWN, D), f32),
        pltpu.VMEM((SCAN_CH,), i32),
        pltpu.VMEM((SCAN_CH,), i32),
        pltpu.VMEM((CBUF,), i32),
        pltpu.VMEM((CBUF,), i32),
        pltpu.VMEM((CBUF,), i32),
        pltpu.VMEM((CBUF,), i32),
        pltpu.VMEM((SB,), i32),
        pltpu.VMEM((SB,), i32),
        pltpu.VMEM((SB, D), f32),
        pltpu.VMEM((SB, D), f32),
        pltpu.SemaphoreType.DMA,
    ],
)(_agg1_body)


N_EDGES_P = 160256
PTE = N_EDGES_P // NT
EDGE_CH = 112
EDGE_FULL = 44
EDGE_TAIL = 80


def _sc_edge_vec_body(qh, qt, re, rel0, ht0, ht1, rr,
                      v_out,
                      i0, i1, ir, i0t, i1t, irt, bqh, bqt, bre, br0, sem):
  c = lax.axis_index("c")
  s = lax.axis_index("s")
  wid = s * NC + c
  tile_base = wid * PTE

  def chunk(base, ch, bi0, bi1, bir):
    l0 = pltpu.async_copy(ht0.at[pl.ds(base, ch)], bi0, sem)
    l1 = pltpu.async_copy(ht1.at[pl.ds(base, ch)], bi1, sem)
    l2 = pltpu.async_copy(rr.at[pl.ds(base, ch)], bir, sem)
    l0.wait()
    l1.wait()
    l2.wait()
    g0 = pltpu.async_copy(qh.at[bi0], bqh.at[pl.ds(0, ch)], sem)
    g1 = pltpu.async_copy(qt.at[bi1], bqt.at[pl.ds(0, ch)], sem)
    g2 = pltpu.async_copy(re.at[bir], bre.at[pl.ds(0, ch)], sem)
    g3 = pltpu.async_copy(rel0.at[bir], br0.at[pl.ds(0, ch)], sem)
    g0.wait()
    g1.wait()
    g2.wait()
    g3.wait()

    def row_body(i, rcarry):
      for j in range(D // L):
        sl = pl.ds(j * L, L)
        q = bqh[i, sl] + bqt[i, sl] + bre[i, sl]
        bqh[i, sl] = jnp.where(q > 0, q, 0.01 * q) + br0[i, sl]
      return rcarry

    lax.fori_loop(0, ch, row_body, 0)
    pltpu.sync_copy(bqh.at[pl.ds(0, ch)], v_out.at[pl.ds(base, ch)])

  def loop_body(g, carry):
    chunk(tile_base + g * EDGE_CH, EDGE_CH, i0, i1, ir)
    return carry

  lax.fori_loop(0, EDGE_FULL, loop_body, 0)
  chunk(tile_base + EDGE_FULL * EDGE_CH, EDGE_TAIL, i0t, i1t, irt)


_sc_edge_vec = functools.partial(
    pl.kernel,
    out_type=jax.ShapeDtypeStruct((N_EDGES_P, D), f32),
    mesh=_mesh,
    compiler_params=pltpu.CompilerParams(needs_layout_passes=False),
    scratch_types=[
        pltpu.VMEM((EDGE_CH,), i32),
        pltpu.VMEM((EDGE_CH,), i32),
        pltpu.VMEM((EDGE_CH,), i32),
        pltpu.VMEM((EDGE_TAIL,), i32),
        pltpu.VMEM((EDGE_TAIL,), i32),
        pltpu.VMEM((EDGE_TAIL,), i32),
        pltpu.VMEM((EDGE_CH, D), f32),
        pltpu.VMEM((EDGE_CH, D), f32),
        pltpu.VMEM((EDGE_CH, D), f32),
        pltpu.VMEM((EDGE_CH, D), f32),
        pltpu.SemaphoreType.DMA,
    ],
)(_sc_edge_vec_body)


def _sc_decoder_body(h2, relv, ht0, ht1, rr,
                     ssq_out,
                     i0, i1, ir, i0t, i1t, irt, bh, bt, brv, bout, sem):
  c = lax.axis_index("c")
  s = lax.axis_index("s")
  wid = s * NC + c
  tile_base = wid * PTE
  lanes = lax.iota(i32, L)

  def chunk(base, ch, bi0, bi1, bir):
    l0 = pltpu.async_copy(ht0.at[pl.ds(base, ch)], bi0, sem)
    l1 = pltpu.async_copy(ht1.at[pl.ds(base, ch)], bi1, sem)
    l2 = pltpu.async_copy(rr.at[pl.ds(base, ch)], bir, sem)
    l0.wait()
    l1.wait()
    l2.wait()
    g0 = pltpu.async_copy(h2.at[bi0], bh.at[pl.ds(0, ch)], sem)
    g1 = pltpu.async_copy(h2.at[bi1], bt.at[pl.ds(0, ch)], sem)
    g2 = pltpu.async_copy(relv.at[bir], brv.at[pl.ds(0, ch)], sem)
    g0.wait()
    g1.wait()
    g2.wait()

    def grp_body(g2, gcarry):
      def edge_body(i, outv):
        e = g2 * L + i
        vacc = jnp.zeros((L,), f32)
        for j in range(D // L):
          sl = pl.ds(j * L, L)
          t = bh[e, sl] + brv[e, sl] - bt[e, sl]
          vacc = vacc + t * t
        return jnp.where(lanes == i, jnp.sum(vacc), outv)

      outv = lax.fori_loop(0, L, edge_body, jnp.zeros((L,), f32))
      bout[pl.ds(g2 * L, L)] = outv
      return gcarry

    lax.fori_loop(0, ch // L, grp_body, 0)
    pltpu.sync_copy(bout.at[pl.ds(0, ch)], ssq_out.at[pl.ds(base, ch)])

  def loop_body(g, carry):
    chunk(tile_base + g * EDGE_CH, EDGE_CH, i0, i1, ir)
    return carry

  lax.fori_loop(0, EDGE_FULL, loop_body, 0)
  chunk(tile_base + EDGE_FULL * EDGE_CH, EDGE_TAIL, i0t, i1t, irt)


_sc_decoder = functools.partial(
    pl.kernel,
    out_type=jax.ShapeDtypeStruct((N_EDGES_P,), f32),
    mesh=_mesh,
    compiler_params=pltpu.CompilerParams(needs_layout_passes=False),
    scratch_types=[
        pltpu.VMEM((EDGE_CH,), i32),
        pltpu.VMEM((EDGE_CH,), i32),
        pltpu.VMEM((EDGE_CH,), i32),
        pltpu.VMEM((EDGE_TAIL,), i32),
        pltpu.VMEM((EDGE_TAIL,), i32),
        pltpu.VMEM((EDGE_TAIL,), i32),
        pltpu.VMEM((EDGE_CH, D), f32),
        pltpu.VMEM((EDGE_CH, D), f32),
        pltpu.VMEM((EDGE_CH, D), f32),
        pltpu.VMEM((EDGE_CH,), f32),
        pltpu.SemaphoreType.DMA,
    ],
)(_sc_decoder_body)


def kernel(entity_feat, W_in, b_in, rel_emb, msgF_W, msgF_b, msgB_W, msgB_b,
           mp_g, mp_b, edge_W, edge_b, en_g, en_b, rel_vec, ht, r_tensor,
           queries):
  ht0 = ht[:, 0].astype(i32)
  ht1 = ht[:, 1].astype(i32)
  rr = r_tensor.astype(i32)
  pad = N_EDGES_P - N_EDGES
  ht0p = jnp.pad(ht0, (0, pad))
  ht1p = jnp.pad(ht1, (0, pad))
  rrp = jnp.pad(rr, (0, pad))

  wcat0 = jnp.concatenate([msgF_W[0][:D], msgB_W[0][:D]], axis=1)
  rel_w = jnp.concatenate(
      [msgF_W[0][D:], msgB_W[0][D:], edge_W[0][D:2 * D]], axis=1)
  rel_b = jnp.concatenate([msgF_b[0], msgB_b[0], edge_b[0]])[None]
  rel_pad = jnp.pad(rel_emb, ((0, 12), (0, 0)))
  wcat1 = jnp.concatenate(
      [edge_W[0][:D], edge_W[0][2 * D:], msgF_W[1][:D], msgB_W[1][:D]],
      axis=1)
  w_edge1 = jnp.concatenate([msgF_W[1][D:], msgB_W[1][D:]], axis=1)
  b_edge1 = jnp.concatenate([msgF_b[1], msgB_b[1]])[None]

  h0, p0 = _tc_in_proj(entity_feat, W_in, b_in[None], wcat0)
  pf0 = p0[:, :D]
  pb0 = p0[:, D:]
  rel_t = _tc_matmul_bias(rel_pad, rel_w, rel_b)
  rf0 = rel_t[:, :D]
  rb0 = rel_t[:, D:2 * D]
  re0 = rel_t[:, 2 * D:]

  agg0_p, cnt_p = _sc_agg0(pf0, pb0, rf0, rb0, ht0, ht1, rr)
  agg0 = agg0_p[:N_NODES]
  cnt = cnt_p.reshape(NT, 384)[:, :NOWN].reshape(N_NODES_P)[:N_NODES, None]
  h1, p1 = _tc_node_update(agg0, cnt, h0, mp_g[0][None], mp_b[0][None], wcat1)
  qh = p1[:, :D]
  qt = p1[:, D:2 * D]
  pf1 = p1[:, 2 * D:3 * D]
  pb1 = p1[:, 3 * D:]

  v = _sc_edge_vec(qh, qt, re0, rel_emb, ht0p, ht1p, rrp)[:N_EDGES]

  m = _tc_edge_mm(v, en_g[0][None], en_b[0][None], w_edge1, b_edge1)
  mf = m[:, :D]
  mb = m[:, D:]

  agg1_p = _sc_agg1(pf1, pb1, mf, mb, ht0, ht1)
  agg1 = agg1_p[:N_NODES]
  h2 = _tc_node_final(agg1, cnt, h1, mp_g[1][None], mp_b[1][None])

  ssq = _sc_decoder(h2, rel_vec, ht0p, ht1p, rrp)[:N_EDGES]
  scores = _tc_score(ssq.reshape(1250, 128),
                     queries.astype(f32).reshape(1250, 128))
  return scores.reshape(N_EDGES)
